# Initial kernel scaffold; baseline (speedup 1.0000x reference)
import jax
import jax.numpy as jnp
from jax.experimental import pallas as pl


def kernel(node_feats, edge_feats, edge_index, edge_attr, Wq, Wk, Wv, Wo, R1, b1, R2, gamma, Wself, Wconv, Rc1, bc1, Rc2):
    raise NotImplementedError("write your pallas kernel here")


# speedup vs baseline: 1.0001x; 1.0001x over previous
import jax
import jax.numpy as jnp
from jax.experimental import pallas as pl

N, E, D, H, L = 10000, 320000, 128, 8, 2
DE = 4
DV = D // 2
DK = D // H
DVH = DV // H
RH = 32


def _ln_body(x_ref, g_ref, o_ref):
    x = x_ref[...]
    mu = jnp.mean(x, axis=-1, keepdims=True)
    var = jnp.mean((x - mu) ** 2, axis=-1, keepdims=True)
    o_ref[...] = (x - mu) / jnp.sqrt(var + 1e-5) * g_ref[...]


def _layernorm(x, gamma):
    return pl.pallas_call(
        _ln_body,
        out_shape=jax.ShapeDtypeStruct(x.shape, x.dtype),
        grid=(10,),
        in_specs=[pl.BlockSpec((N // 10, D), lambda i: (i, 0)),
                  pl.BlockSpec((1, D), lambda i: (0, 0))],
        out_specs=pl.BlockSpec((N // 10, D), lambda i: (i, 0)),
    )(x, gamma.reshape(1, D))


def kernel(node_feats, edge_feats, edge_index, edge_attr, Wq, Wk, Wv, Wo,
           R1, b1, R2, gamma, Wself, Wconv, Rc1, bc1, Rc2):
    src = edge_index[0]
    dst = edge_index[1]
    dist = jnp.linalg.norm(edge_attr, axis=-1, keepdims=True)
    ef = jnp.concatenate([edge_feats, dist], axis=-1)
    x = node_feats
    for l in range(L):
        q = jnp.einsum('nd,dhk->nhk', x, Wq[l])
        k = jnp.einsum('nd,dhk->nhk', x, Wk[l])
        v = jnp.einsum('nd,dhk->nhk', x, Wv[l])
        radial = jax.nn.relu(ef @ R1[l] + b1[l]) @ R2[l]
        ke = jnp.take(k, src, axis=0) * radial[:, :, None]
        logits = jnp.sum(jnp.take(q, dst, axis=0) * ke, axis=-1) / jnp.sqrt(DK)
        m = jax.ops.segment_max(logits, dst, num_segments=N)
        m = jnp.where(jnp.isfinite(m), m, 0.0)
        p = jnp.exp(logits - jnp.take(m, dst, axis=0))
        denom = jax.ops.segment_sum(p, dst, num_segments=N)
        attn = p / (jnp.take(denom, dst, axis=0) + 1e-9)
        agg = jax.ops.segment_sum(attn[:, :, None] * jnp.take(v, src, axis=0), dst, num_segments=N)
        x = agg.reshape(N, DV) @ Wo[l]
        x = _layernorm(x, gamma[l])
    radial_c = jax.nn.relu(ef @ Rc1 + bc1) @ Rc2
    msg = radial_c * (jnp.take(x, src, axis=0) @ Wconv)
    out = x @ Wself + jax.ops.segment_sum(msg, dst, num_segments=N)
    return out


# speedup vs baseline: 21.6440x; 21.6409x over previous
import functools

import jax
import jax.numpy as jnp
from jax import lax
from jax.experimental import pallas as pl
from jax.experimental.pallas import tpu as pltpu
from jax.experimental.pallas import tpu_sc as plsc

N, E, D, H = 10000, 320000, 128, 8
DE = 4
DV = D // 2
DK = D // H
DVH = DV // H
RH = 32
LAYERS = 2

NC, NS = 2, 16
NW = NC * NS
EPW = E // NW
C = 128
NFULL = EPW // C
REM = EPW - NFULL * C
ACCW = 80
RPT = 632
RPT_LAST = N - (NS - 1) * RPT

BN = 1000
BE = 8000


def _qkv_body(x_ref, wq_ref, wk_ref, wv_ref, q_ref, k_ref, v_ref):
    x = x_ref[...]
    q_ref[...] = jnp.dot(x, wq_ref[...], preferred_element_type=jnp.float32) * 0.25
    k_ref[...] = jnp.dot(x, wk_ref[...], preferred_element_type=jnp.float32)
    v_ref[...] = jnp.dot(x, wv_ref[...], preferred_element_type=jnp.float32)


def _qkv(x, wq, wk, wv):
    return pl.pallas_call(
        _qkv_body,
        grid=(N // BN,),
        in_specs=[
            pl.BlockSpec((BN, D), lambda i: (i, 0)),
            pl.BlockSpec((D, D), lambda i: (0, 0)),
            pl.BlockSpec((D, D), lambda i: (0, 0)),
            pl.BlockSpec((D, DV), lambda i: (0, 0)),
        ],
        out_specs=[
            pl.BlockSpec((BN, D), lambda i: (i, 0)),
            pl.BlockSpec((BN, D), lambda i: (i, 0)),
            pl.BlockSpec((BN, DV), lambda i: (i, 0)),
        ],
        out_shape=[
            jax.ShapeDtypeStruct((N, D), jnp.float32),
            jax.ShapeDtypeStruct((N, D), jnp.float32),
            jax.ShapeDtypeStruct((N, DV), jnp.float32),
        ],
    )(x, wq, wk, wv)


def _radial_body(efe_ref, ea_ref, r1a_ref, r2a_ref, r1b_ref, r2b_ref,
                 rc1_ref, rc2_ref, b1a_ref, b1b_ref, bc1_ref,
                 ra_ref, rb_ref, rc_ref):
    fe = efe_ref[...]
    ea = ea_ref[...]
    dist = jnp.sqrt(jnp.sum(ea * ea, axis=1, keepdims=True))
    ef = jnp.concatenate([fe, dist], axis=1)

    ha = jnp.maximum(jnp.dot(ef, r1a_ref[...], preferred_element_type=jnp.float32)
                     + b1a_ref[...], 0.0)
    ra_ref[...] = jnp.dot(ha, r2a_ref[...], preferred_element_type=jnp.float32)
    hb = jnp.maximum(jnp.dot(ef, r1b_ref[...], preferred_element_type=jnp.float32)
                     + b1b_ref[...], 0.0)
    rb_ref[...] = jnp.dot(hb, r2b_ref[...], preferred_element_type=jnp.float32)
    hc = jnp.maximum(jnp.dot(ef, rc1_ref[...], preferred_element_type=jnp.float32)
                     + bc1_ref[...], 0.0)
    rcv = jnp.dot(hc, rc2_ref[...], preferred_element_type=jnp.float32)
    rc_ref[...] = rcv * jnp.ones((1, H), jnp.float32)


def _radials(edge_feats, edge_attr, R1, b1, R2, Rc1, bc1, Rc2):
    full = lambda shape: pl.BlockSpec(shape, lambda i: (0, 0))
    return pl.pallas_call(
        _radial_body,
        grid=(E // BE,),
        in_specs=[
            pl.BlockSpec((BE, DE), lambda i: (i, 0)),
            pl.BlockSpec((BE, 3), lambda i: (i, 0)),
            full((DE + 1, RH)), full((RH, H)),
            full((DE + 1, RH)), full((RH, H)),
            full((DE + 1, RH)), full((RH, 1)),
            full((1, RH)), full((1, RH)), full((1, RH)),
        ],
        out_specs=[
            pl.BlockSpec((BE, H), lambda i: (i, 0)),
            pl.BlockSpec((BE, H), lambda i: (i, 0)),
            pl.BlockSpec((BE, H), lambda i: (i, 0)),
        ],
        out_shape=[
            jax.ShapeDtypeStruct((E, H), jnp.float32),
            jax.ShapeDtypeStruct((E, H), jnp.float32),
            jax.ShapeDtypeStruct((E, H), jnp.float32),
        ],
    )(edge_feats, edge_attr, R1[0], R2[0], R1[1], R2[1], Rc1, Rc2,
      b1[0].reshape(1, RH), b1[1].reshape(1, RH), bc1.reshape(1, RH))


def _epilogue_body(acc_ref, wo_ref, g_ref, x_ref):
    a = acc_ref[0] + acc_ref[1]
    aggv = a[:, 0:DV]
    denom = a[:, DV:DV + H]
    r8 = lax.broadcasted_iota(jnp.int32, (H, DV), 0)
    c64 = lax.broadcasted_iota(jnp.int32, (H, DV), 1)
    sel = (c64 // DVH == r8).astype(jnp.float32)
    scale = jnp.dot(1.0 / (denom + 1e-9), sel, preferred_element_type=jnp.float32)
    x = jnp.dot(aggv * scale, wo_ref[...], preferred_element_type=jnp.float32)
    mu = jnp.mean(x, axis=1, keepdims=True)
    var = jnp.mean((x - mu) * (x - mu), axis=1, keepdims=True)
    x_ref[...] = (x - mu) / jnp.sqrt(var + 1e-5) * g_ref[...]


def _epilogue(acc, wo, gamma):
    return pl.pallas_call(
        _epilogue_body,
        grid=(N // BN,),
        in_specs=[
            pl.BlockSpec((NC, BN, ACCW), lambda i: (0, i, 0)),
            pl.BlockSpec((DV, D), lambda i: (0, 0)),
            pl.BlockSpec((1, D), lambda i: (0, 0)),
        ],
        out_specs=pl.BlockSpec((BN, D), lambda i: (i, 0)),
        out_shape=jax.ShapeDtypeStruct((N, D), jnp.float32),
    )(acc, wo, gamma.reshape(1, D))


def _finalpre_body(x_ref, ws_ref, wc_ref, xs_ref, y_ref):
    x = x_ref[...]
    xs_ref[...] = jnp.dot(x, ws_ref[...], preferred_element_type=jnp.float32)
    y_ref[...] = jnp.dot(x, wc_ref[...], preferred_element_type=jnp.float32)


def _finalpre(x, wself, wconv):
    return pl.pallas_call(
        _finalpre_body,
        grid=(N // BN,),
        in_specs=[
            pl.BlockSpec((BN, D), lambda i: (i, 0)),
            pl.BlockSpec((D, D), lambda i: (0, 0)),
            pl.BlockSpec((D, D), lambda i: (0, 0)),
        ],
        out_specs=[
            pl.BlockSpec((BN, D), lambda i: (i, 0)),
            pl.BlockSpec((BN, D), lambda i: (i, 0)),
        ],
        out_shape=[
            jax.ShapeDtypeStruct((N, D), jnp.float32),
            jax.ShapeDtypeStruct((N, D), jnp.float32),
        ],
    )(x, wself, wconv)


def _finaladd_body(xs_ref, ca_ref, o_ref):
    o_ref[...] = xs_ref[...] + ca_ref[0] + ca_ref[1]


def _finaladd(xs, ca):
    return pl.pallas_call(
        _finaladd_body,
        grid=(N // BN,),
        in_specs=[
            pl.BlockSpec((BN, D), lambda i: (i, 0)),
            pl.BlockSpec((NC, BN, D), lambda i: (0, i, 0)),
        ],
        out_specs=pl.BlockSpec((BN, D), lambda i: (i, 0)),
        out_shape=jax.ShapeDtypeStruct((N, D), jnp.float32),
    )(xs, ca)


_MESH = plsc.VectorSubcoreMesh(core_axis_name="c", subcore_axis_name="s",
                               num_cores=NC, num_subcores=NS)


def _per_subcore_rows(sid, fn):
    @pl.when(sid < NS - 1)
    def _():
        fn(sid * RPT, RPT)

    @pl.when(sid == NS - 1)
    def _():
        fn((NS - 1) * RPT, RPT_LAST)


def _edge_attn_kernel(q_hbm, k_hbm, v_hbm, rad_hbm, src_hbm, dst_hbm, z_hbm,
                      out_hbm, acc, sidx, didx, sidx2, didx2,
                      qrow, krow, vrow, radf, logit, pbuf, pv, sq, sk, sv, sr):
    cid = lax.axis_index("c")
    sid = lax.axis_index("s")
    wid = sid * NC + cid

    lane = lax.broadcasted_iota(jnp.int32, (16,), 0)
    lane_lo = lane < 8
    lo_f = jnp.where(lane_lo, 1.0, 0.0).astype(jnp.float32)
    pbase = jnp.where(lane_lo, 0, 1)
    lane15 = lane == 15

    _per_subcore_rows(sid, lambda st, cnt: pltpu.sync_copy(
        z_hbm.at[pl.ds(st, cnt)], acc.at[pl.ds(st, cnt)]))

    pbuf[pl.ds(C * H - 8, 16)] = jnp.zeros((16,), jnp.float32)

    plsc.subcore_barrier()

    def _chunk(eo, cc, sidx_r, didx_r):
        pltpu.sync_copy(src_hbm.at[pl.ds(eo, cc)], sidx_r)
        pltpu.sync_copy(dst_hbm.at[pl.ds(eo, cc)], didx_r)
        cq = pltpu.async_copy(q_hbm.at[didx_r], qrow.at[pl.ds(0, cc)], sq)
        ck = pltpu.async_copy(k_hbm.at[sidx_r], krow.at[pl.ds(0, cc)], sk)
        cv = pltpu.async_copy(v_hbm.at[sidx_r], vrow.at[pl.ds(0, cc)], sv)
        cr = pltpu.async_copy(rad_hbm.at[pl.ds(eo * H, cc * H)],
                              radf.at[pl.ds(0, cc * H)], sr)
        cq.wait()
        ck.wait()
        cv.wait()
        cr.wait()

        def _dot(e, carry):
            for h in range(H):
                qv = qrow[e, pl.ds(h * DK, DK)]
                kv = krow[e, pl.ds(h * DK, DK)]
                s = plsc.cumsum(qv * kv)
                plsc.store_scatter(logit, [jnp.full((16,), e * H + h, jnp.int32)],
                                   s, mask=lane15)
            return carry
        lax.fori_loop(0, cc, _dot, 0)

        def _pexp(i, carry):
            lv = logit[pl.ds(i * 16, 16)]
            rv = radf[pl.ds(i * 16, 16)]
            pbuf[pl.ds(i * 16, 16)] = jnp.exp(lv * rv)
            return carry
        lax.fori_loop(0, cc // 2, _pexp, 0)

        def _pv(e, carry):
            pe = pbuf[pl.ds(e * H, 16)] * lo_f
            pv[e, pl.ds(DV, 16)] = pe
            for j in range(DV // 16):
                vv = vrow[e, pl.ds(j * 16, 16)]
                pvec = plsc.load_gather(
                    pbuf, [jnp.full((16,), e * H + 2 * j, jnp.int32) + pbase])
                pv[e, pl.ds(j * 16, 16)] = vv * pvec
            return carry
        lax.fori_loop(0, cc, _pv, 0)

        pltpu.sync_copy(pv.at[pl.ds(0, cc)], acc.at[didx_r], add=True)

    def _full_chunk(ci, carry):
        eo = pl.multiple_of(wid * EPW + ci * C, 8)
        _chunk(eo, C, sidx, didx)
        return carry
    lax.fori_loop(0, NFULL, _full_chunk, 0)
    _chunk(pl.multiple_of(wid * EPW + NFULL * C, 8), REM, sidx2, didx2)

    plsc.subcore_barrier()
    _per_subcore_rows(sid, lambda st, cnt: pltpu.sync_copy(
        acc.at[pl.ds(st, cnt)], out_hbm.at[pl.ds(cid * N + st, cnt)]))


def _edge_attn(q, k, v, radf, src, dst, zeros80):
    f = pl.kernel(
        _edge_attn_kernel,
        out_type=jax.ShapeDtypeStruct((NC * N, ACCW), jnp.float32),
        mesh=_MESH,
        compiler_params=pltpu.CompilerParams(needs_layout_passes=False, use_tc_tiling_on_sc=False),
        scratch_types=[
            pltpu.VMEM_SHARED((N, ACCW), jnp.float32),
            pltpu.VMEM((C,), jnp.int32),
            pltpu.VMEM((C,), jnp.int32),
            pltpu.VMEM((REM,), jnp.int32),
            pltpu.VMEM((REM,), jnp.int32),
            pltpu.VMEM((C, D), jnp.float32),
            pltpu.VMEM((C, D), jnp.float32),
            pltpu.VMEM((C, DV), jnp.float32),
            pltpu.VMEM((C * H,), jnp.float32),
            pltpu.VMEM((C * H + 8,), jnp.float32),
            pltpu.VMEM((C * H + 8,), jnp.float32),
            pltpu.VMEM((C, ACCW), jnp.float32),
            pltpu.SemaphoreType.DMA,
            pltpu.SemaphoreType.DMA,
            pltpu.SemaphoreType.DMA,
            pltpu.SemaphoreType.DMA,
        ],
    )
    return f(q, k, v, radf, src, dst, zeros80)


def _conv_kernel(y_hbm, rc_hbm, src_hbm, dst_hbm, z_hbm, out_hbm,
                 acc, sidx, didx, sidx2, didx2, yrow, rcf, msg, sy, sr):
    cid = lax.axis_index("c")
    sid = lax.axis_index("s")
    wid = sid * NC + cid

    _per_subcore_rows(sid, lambda st, cnt: pltpu.sync_copy(
        z_hbm.at[pl.ds(st, cnt)], acc.at[pl.ds(st, cnt)]))
    plsc.subcore_barrier()

    def _chunk(eo, cc, sidx_r, didx_r):
        pltpu.sync_copy(src_hbm.at[pl.ds(eo, cc)], sidx_r)
        pltpu.sync_copy(dst_hbm.at[pl.ds(eo, cc)], didx_r)
        cy = pltpu.async_copy(y_hbm.at[sidx_r], yrow.at[pl.ds(0, cc)], sy)
        cr = pltpu.async_copy(rc_hbm.at[pl.ds(eo * H, cc * H)],
                              rcf.at[pl.ds(0, cc * H)], sr)
        cy.wait()
        cr.wait()

        def _scale(e, carry):
            bv = plsc.load_gather(rcf, [jnp.full((16,), e * H, jnp.int32)])
            for j in range(D // 16):
                msg[e, pl.ds(j * 16, 16)] = yrow[e, pl.ds(j * 16, 16)] * bv
            return carry
        lax.fori_loop(0, cc, _scale, 0)

        pltpu.sync_copy(msg.at[pl.ds(0, cc)], acc.at[didx_r], add=True)

    def _full_chunk(ci, carry):
        eo = pl.multiple_of(wid * EPW + ci * C, 8)
        _chunk(eo, C, sidx, didx)
        return carry
    lax.fori_loop(0, NFULL, _full_chunk, 0)
    _chunk(pl.multiple_of(wid * EPW + NFULL * C, 8), REM, sidx2, didx2)

    plsc.subcore_barrier()
    _per_subcore_rows(sid, lambda st, cnt: pltpu.sync_copy(
        acc.at[pl.ds(st, cnt)], out_hbm.at[pl.ds(cid * N + st, cnt)]))


def _conv(y, rcf, src, dst, zeros128):
    f = pl.kernel(
        _conv_kernel,
        out_type=jax.ShapeDtypeStruct((NC * N, D), jnp.float32),
        mesh=_MESH,
        compiler_params=pltpu.CompilerParams(needs_layout_passes=False, use_tc_tiling_on_sc=False),
        scratch_types=[
            pltpu.VMEM_SHARED((N, D), jnp.float32),
            pltpu.VMEM((C,), jnp.int32),
            pltpu.VMEM((C,), jnp.int32),
            pltpu.VMEM((REM,), jnp.int32),
            pltpu.VMEM((REM,), jnp.int32),
            pltpu.VMEM((C, D), jnp.float32),
            pltpu.VMEM((C * H,), jnp.float32),
            pltpu.VMEM((C, D), jnp.float32),
            pltpu.SemaphoreType.DMA,
            pltpu.SemaphoreType.DMA,
        ],
    )
    return f(y, rcf, src, dst, zeros128)


def kernel(node_feats, edge_feats, edge_index, edge_attr, Wq, Wk, Wv, Wo,
           R1, b1, R2, gamma, Wself, Wconv, Rc1, bc1, Rc2):
    src = edge_index[0].astype(jnp.int32)
    dst = edge_index[1].astype(jnp.int32)

    ra0, ra1, rc8 = _radials(edge_feats, edge_attr, R1, b1, R2, Rc1, bc1, Rc2)
    radfs = (ra0.reshape(E * H), ra1.reshape(E * H))
    rcf = rc8.reshape(E * H)

    zeros80 = jnp.zeros((N, ACCW), jnp.float32)
    zeros128 = jnp.zeros((N, D), jnp.float32)

    x = node_feats
    for l in range(LAYERS):
        q, k, v = _qkv(x, Wq[l].reshape(D, D), Wk[l].reshape(D, D),
                       Wv[l].reshape(D, DV))
        acc = _edge_attn(q, k, v, radfs[l], src, dst, zeros80)
        x = _epilogue(acc.reshape(NC, N, ACCW), Wo[l], gamma[l])

    xs, y = _finalpre(x, Wself, Wconv)
    ca = _conv(y, rcf, src, dst, zeros128)
    return _finaladd(xs, ca.reshape(NC, N, D))


# speedup vs baseline: 49.7853x; 2.3002x over previous
import functools

import jax
import jax.numpy as jnp
from jax import lax
from jax.experimental import pallas as pl
from jax.experimental.pallas import tpu as pltpu
from jax.experimental.pallas import tpu_sc as plsc

N, E, D, H = 10000, 320000, 128, 8
DE = 4
DV = D // 2
DK = D // H
DVH = DV // H
RH = 32
LAYERS = 2

NC, NS = 2, 16
NW = NC * NS
EPW = E // NW
C = 64
NFULL = EPW // C
REM = EPW - NFULL * C
ACCW = 80
RPT = 632
RPT_LAST = N - (NS - 1) * RPT

BN = 1000
BE = 8000


def _qkv_body(x_ref, wq_ref, wk_ref, wv_ref, q_ref, k_ref, v_ref):
    x = x_ref[...]
    q_ref[...] = jnp.dot(x, wq_ref[...], preferred_element_type=jnp.float32) * 0.25
    k_ref[...] = jnp.dot(x, wk_ref[...], preferred_element_type=jnp.float32)
    v_ref[...] = jnp.dot(x, wv_ref[...], preferred_element_type=jnp.float32)


def _qkv(x, wq, wk, wv):
    return pl.pallas_call(
        _qkv_body,
        grid=(N // BN,),
        in_specs=[
            pl.BlockSpec((BN, D), lambda i: (i, 0)),
            pl.BlockSpec((D, D), lambda i: (0, 0)),
            pl.BlockSpec((D, D), lambda i: (0, 0)),
            pl.BlockSpec((D, DV), lambda i: (0, 0)),
        ],
        out_specs=[
            pl.BlockSpec((BN, D), lambda i: (i, 0)),
            pl.BlockSpec((BN, D), lambda i: (i, 0)),
            pl.BlockSpec((BN, DV), lambda i: (i, 0)),
        ],
        out_shape=[
            jax.ShapeDtypeStruct((N, D), jnp.float32),
            jax.ShapeDtypeStruct((N, D), jnp.float32),
            jax.ShapeDtypeStruct((N, DV), jnp.float32),
        ],
    )(x, wq, wk, wv)


def _radial_body(efe_ref, ea_ref, r1a_ref, r2a_ref, r1b_ref, r2b_ref,
                 rc1_ref, rc2_ref, b1a_ref, b1b_ref, bc1_ref,
                 ra_ref, rb_ref, rc_ref):
    fe = efe_ref[...]
    ea = ea_ref[...]
    dist = jnp.sqrt(jnp.sum(ea * ea, axis=1, keepdims=True))
    ef = jnp.concatenate([fe, dist], axis=1)

    ha = jnp.maximum(jnp.dot(ef, r1a_ref[...], preferred_element_type=jnp.float32)
                     + b1a_ref[...], 0.0)
    ra_ref[...] = jnp.dot(ha, r2a_ref[...], preferred_element_type=jnp.float32)
    hb = jnp.maximum(jnp.dot(ef, r1b_ref[...], preferred_element_type=jnp.float32)
                     + b1b_ref[...], 0.0)
    rb_ref[...] = jnp.dot(hb, r2b_ref[...], preferred_element_type=jnp.float32)
    hc = jnp.maximum(jnp.dot(ef, rc1_ref[...], preferred_element_type=jnp.float32)
                     + bc1_ref[...], 0.0)
    rcv = jnp.dot(hc, rc2_ref[...], preferred_element_type=jnp.float32)
    rc_ref[...] = rcv * jnp.ones((1, H), jnp.float32)


def _radials(edge_feats, edge_attr, R1, b1, R2, Rc1, bc1, Rc2):
    full = lambda shape: pl.BlockSpec(shape, lambda i: (0, 0))
    return pl.pallas_call(
        _radial_body,
        grid=(E // BE,),
        in_specs=[
            pl.BlockSpec((BE, DE), lambda i: (i, 0)),
            pl.BlockSpec((BE, 3), lambda i: (i, 0)),
            full((DE + 1, RH)), full((RH, H)),
            full((DE + 1, RH)), full((RH, H)),
            full((DE + 1, RH)), full((RH, 1)),
            full((1, RH)), full((1, RH)), full((1, RH)),
        ],
        out_specs=[
            pl.BlockSpec((BE, H), lambda i: (i, 0)),
            pl.BlockSpec((BE, H), lambda i: (i, 0)),
            pl.BlockSpec((BE, H), lambda i: (i, 0)),
        ],
        out_shape=[
            jax.ShapeDtypeStruct((E, H), jnp.float32),
            jax.ShapeDtypeStruct((E, H), jnp.float32),
            jax.ShapeDtypeStruct((E, H), jnp.float32),
        ],
    )(edge_feats, edge_attr, R1[0], R2[0], R1[1], R2[1], Rc1, Rc2,
      b1[0].reshape(1, RH), b1[1].reshape(1, RH), bc1.reshape(1, RH))


def _epilogue_body(acc_ref, wo_ref, g_ref, x_ref):
    a = acc_ref[0] + acc_ref[1]
    aggv = a[:, 0:DV]
    denom = a[:, DV:DV + H]
    r8 = lax.broadcasted_iota(jnp.int32, (H, DV), 0)
    c64 = lax.broadcasted_iota(jnp.int32, (H, DV), 1)
    sel = (c64 // DVH == r8).astype(jnp.float32)
    scale = jnp.dot(1.0 / (denom + 1e-9), sel, preferred_element_type=jnp.float32)
    x = jnp.dot(aggv * scale, wo_ref[...], preferred_element_type=jnp.float32)
    mu = jnp.mean(x, axis=1, keepdims=True)
    var = jnp.mean((x - mu) * (x - mu), axis=1, keepdims=True)
    x_ref[...] = (x - mu) / jnp.sqrt(var + 1e-5) * g_ref[...]


def _epilogue(acc, wo, gamma):
    return pl.pallas_call(
        _epilogue_body,
        grid=(N // BN,),
        in_specs=[
            pl.BlockSpec((NC, BN, ACCW), lambda i: (0, i, 0)),
            pl.BlockSpec((DV, D), lambda i: (0, 0)),
            pl.BlockSpec((1, D), lambda i: (0, 0)),
        ],
        out_specs=pl.BlockSpec((BN, D), lambda i: (i, 0)),
        out_shape=jax.ShapeDtypeStruct((N, D), jnp.float32),
    )(acc, wo, gamma.reshape(1, D))


def _finalpre_body(x_ref, ws_ref, wc_ref, xs_ref, y_ref):
    x = x_ref[...]
    xs_ref[...] = jnp.dot(x, ws_ref[...], preferred_element_type=jnp.float32)
    y_ref[...] = jnp.dot(x, wc_ref[...], preferred_element_type=jnp.float32)


def _finalpre(x, wself, wconv):
    return pl.pallas_call(
        _finalpre_body,
        grid=(N // BN,),
        in_specs=[
            pl.BlockSpec((BN, D), lambda i: (i, 0)),
            pl.BlockSpec((D, D), lambda i: (0, 0)),
            pl.BlockSpec((D, D), lambda i: (0, 0)),
        ],
        out_specs=[
            pl.BlockSpec((BN, D), lambda i: (i, 0)),
            pl.BlockSpec((BN, D), lambda i: (i, 0)),
        ],
        out_shape=[
            jax.ShapeDtypeStruct((N, D), jnp.float32),
            jax.ShapeDtypeStruct((N, D), jnp.float32),
        ],
    )(x, wself, wconv)


def _finaladd_body(xs_ref, ca_ref, o_ref):
    o_ref[...] = xs_ref[...] + ca_ref[0] + ca_ref[1]


def _finaladd(xs, ca):
    return pl.pallas_call(
        _finaladd_body,
        grid=(N // BN,),
        in_specs=[
            pl.BlockSpec((BN, D), lambda i: (i, 0)),
            pl.BlockSpec((NC, BN, D), lambda i: (0, i, 0)),
        ],
        out_specs=pl.BlockSpec((BN, D), lambda i: (i, 0)),
        out_shape=jax.ShapeDtypeStruct((N, D), jnp.float32),
    )(xs, ca)


_MESH = plsc.VectorSubcoreMesh(core_axis_name="c", subcore_axis_name="s",
                               num_cores=NC, num_subcores=NS)


def _per_subcore_rows(sid, fn):
    @pl.when(sid < NS - 1)
    def _():
        fn(sid * RPT, RPT)

    @pl.when(sid == NS - 1)
    def _():
        fn((NS - 1) * RPT, RPT_LAST)


def _edge_attn_kernel(q_hbm, k_hbm, v_hbm, rad_hbm, src_hbm, dst_hbm, z_hbm,
                      out_hbm, acc,
                      sidx0, didx0, sidx1, didx1, sidx2, didx2,
                      qrow0, krow0, vrow0, radf0,
                      qrow1, krow1, vrow1, radf1,
                      logit, pbuf, pv,
                      sq0, sk0, sv0, sr0, sq1, sk1, sv1, sr1):
    cid = lax.axis_index("c")
    sid = lax.axis_index("s")
    ebase = (sid * NC + cid) * EPW

    lane = lax.broadcasted_iota(jnp.int32, (16,), 0)
    lane_lo = lane < 8
    lo_f = jnp.where(lane_lo, 1.0, 0.0).astype(jnp.float32)
    lane15 = lane == 15
    pbase = jnp.where(lane_lo, 0, 1)

    _per_subcore_rows(sid, lambda st, cnt: pltpu.sync_copy(
        z_hbm.at[pl.ds(st, cnt)], acc.at[pl.ds(st, cnt)]))

    pbuf[pl.ds(C * H - 8, 16)] = jnp.zeros((16,), jnp.float32)

    plsc.subcore_barrier()

    bufs = ((sidx0, didx0, qrow0, krow0, vrow0, radf0, sq0, sk0, sv0, sr0),
            (sidx1, didx1, qrow1, krow1, vrow1, radf1, sq1, sk1, sv1, sr1))

    def issue(g, b):
        si, di, qr, kr, vr, rf, sq, sk, sv, sr = bufs[b]
        eo = pl.multiple_of(ebase + g * C, 8)
        pltpu.sync_copy(src_hbm.at[pl.ds(eo, C)], si)
        pltpu.sync_copy(dst_hbm.at[pl.ds(eo, C)], di)
        pltpu.async_copy(q_hbm.at[di], qr, sq)
        pltpu.async_copy(k_hbm.at[si], kr, sk)
        pltpu.async_copy(v_hbm.at[si], vr, sv)
        pltpu.async_copy(rad_hbm.at[pl.ds(eo * H, C * H)], rf, sr)

    def wait(b):
        si, di, qr, kr, vr, rf, sq, sk, sv, sr = bufs[b]
        pltpu.make_async_copy(q_hbm.at[di], qr, sq).wait()
        pltpu.make_async_copy(k_hbm.at[si], kr, sk).wait()
        pltpu.make_async_copy(v_hbm.at[si], vr, sv).wait()
        pltpu.make_async_copy(rad_hbm.at[pl.ds(0, C * H)], rf, sr).wait()

    def compute(b, cc):
        si, di, qr, kr, vr, rf, *_ = bufs[b]

        @plsc.parallel_loop(0, cc // 2, unroll=2)
        def _pair(i):
            for t in range(2):
                e = 2 * i + t
                for h in range(H):
                    qv = qr[e, pl.ds(h * DK, DK)]
                    kv = kr[e, pl.ds(h * DK, DK)]
                    s = plsc.cumsum(qv * kv)
                    plsc.store_scatter(
                        logit, [jnp.full((16,), e * H + h, jnp.int32)],
                        s, mask=lane15)
            lv = logit[pl.ds(i * 16, 16)]
            rv = rf[pl.ds(i * 16, 16)]
            pbuf[pl.ds(i * 16, 16)] = jnp.exp(lv * rv)

        @plsc.parallel_loop(0, cc, unroll=2)
        def _pv(e):
            p16 = pbuf[pl.ds(e * H, 16)]
            pv[e, pl.ds(DV, 16)] = p16 * lo_f
            for j in range(DV // 16):
                vv = vr[e, pl.ds(j * 16, 16)]
                pj = plsc.load_gather(
                    pbuf, [jnp.full((16,), e * H + 2 * j, jnp.int32) + pbase])
                pv[e, pl.ds(j * 16, 16)] = vv * pj

        if cc == C:
            pltpu.sync_copy(pv, acc.at[di], add=True)
        else:
            pltpu.sync_copy(pv.at[pl.ds(0, cc)], acc.at[di], add=True)

    issue(0, 0)

    def _body(ci, carry):
        issue(2 * ci + 1, 1)
        wait(0)
        compute(0, C)

        @pl.when(ci < NFULL // 2 - 1)
        def _():
            issue(2 * ci + 2, 0)

        wait(1)
        compute(1, C)
        return carry
    lax.fori_loop(0, NFULL // 2, _body, 0)

    si, di, qr, kr, vr, rf, sq, sk, sv, sr = bufs[0]
    eo = pl.multiple_of(ebase + NFULL * C, 8)
    pltpu.sync_copy(src_hbm.at[pl.ds(eo, REM)], sidx2)
    pltpu.sync_copy(dst_hbm.at[pl.ds(eo, REM)], didx2)
    pltpu.async_copy(q_hbm.at[didx2], qr.at[pl.ds(0, REM)], sq).wait()
    pltpu.async_copy(k_hbm.at[sidx2], kr.at[pl.ds(0, REM)], sk).wait()
    pltpu.async_copy(v_hbm.at[sidx2], vr.at[pl.ds(0, REM)], sv).wait()
    pltpu.async_copy(rad_hbm.at[pl.ds(eo * H, REM * H)],
                     rf.at[pl.ds(0, REM * H)], sr).wait()

    @plsc.parallel_loop(0, REM // 2, unroll=2)
    def _pair_r(i):
        for t in range(2):
            e = 2 * i + t
            for h in range(H):
                s = plsc.cumsum(qr[e, pl.ds(h * DK, DK)] * kr[e, pl.ds(h * DK, DK)])
                plsc.store_scatter(logit, [jnp.full((16,), e * H + h, jnp.int32)],
                                   s, mask=lane15)
        pbuf[pl.ds(i * 16, 16)] = jnp.exp(logit[pl.ds(i * 16, 16)]
                                          * rf[pl.ds(i * 16, 16)])

    @plsc.parallel_loop(0, REM, unroll=2)
    def _pv_r(e):
        p16 = pbuf[pl.ds(e * H, 16)]
        pv[e, pl.ds(DV, 16)] = p16 * lo_f
        for j in range(DV // 16):
            pj = plsc.load_gather(
                pbuf, [jnp.full((16,), e * H + 2 * j, jnp.int32) + pbase])
            pv[e, pl.ds(j * 16, 16)] = vr[e, pl.ds(j * 16, 16)] * pj

    pltpu.sync_copy(pv.at[pl.ds(0, REM)], acc.at[didx2], add=True)

    plsc.subcore_barrier()
    _per_subcore_rows(sid, lambda st, cnt: pltpu.sync_copy(
        acc.at[pl.ds(st, cnt)], out_hbm.at[pl.ds(cid * N + st, cnt)]))


def _edge_attn(q, k, v, radf, src, dst, zeros80):
    f = pl.kernel(
        _edge_attn_kernel,
        out_type=jax.ShapeDtypeStruct((NC * N, ACCW), jnp.float32),
        mesh=_MESH,
        compiler_params=pltpu.CompilerParams(needs_layout_passes=False, use_tc_tiling_on_sc=False),
        scratch_types=[
            pltpu.VMEM_SHARED((N, ACCW), jnp.float32),
            pltpu.VMEM((C,), jnp.int32),
            pltpu.VMEM((C,), jnp.int32),
            pltpu.VMEM((C,), jnp.int32),
            pltpu.VMEM((C,), jnp.int32),
            pltpu.VMEM((REM,), jnp.int32),
            pltpu.VMEM((REM,), jnp.int32),
            pltpu.VMEM((C, D), jnp.float32),
            pltpu.VMEM((C, D), jnp.float32),
            pltpu.VMEM((C, DV), jnp.float32),
            pltpu.VMEM((C * H,), jnp.float32),
            pltpu.VMEM((C, D), jnp.float32),
            pltpu.VMEM((C, D), jnp.float32),
            pltpu.VMEM((C, DV), jnp.float32),
            pltpu.VMEM((C * H,), jnp.float32),
            pltpu.VMEM((C * H + 8,), jnp.float32),
            pltpu.VMEM((C * H + 8,), jnp.float32),
            pltpu.VMEM((C, ACCW), jnp.float32),
            pltpu.SemaphoreType.DMA,
            pltpu.SemaphoreType.DMA,
            pltpu.SemaphoreType.DMA,
            pltpu.SemaphoreType.DMA,
            pltpu.SemaphoreType.DMA,
            pltpu.SemaphoreType.DMA,
            pltpu.SemaphoreType.DMA,
            pltpu.SemaphoreType.DMA,
        ],
    )
    return f(q, k, v, radf, src, dst, zeros80)


def _conv_kernel(y_hbm, rc_hbm, src_hbm, dst_hbm, z_hbm, out_hbm,
                 acc, sidx0, didx0, sidx1, didx1, sidx2, didx2,
                 yrow0, rcf0, yrow1, rcf1, msg,
                 sy0, sr0, sy1, sr1):
    cid = lax.axis_index("c")
    sid = lax.axis_index("s")
    ebase = (sid * NC + cid) * EPW


    _per_subcore_rows(sid, lambda st, cnt: pltpu.sync_copy(
        z_hbm.at[pl.ds(st, cnt)], acc.at[pl.ds(st, cnt)]))
    plsc.subcore_barrier()

    bufs = ((sidx0, didx0, yrow0, rcf0, sy0, sr0),
            (sidx1, didx1, yrow1, rcf1, sy1, sr1))

    def issue(g, b):
        si, di, yr, rf, sy, sr = bufs[b]
        eo = pl.multiple_of(ebase + g * C, 8)
        pltpu.sync_copy(src_hbm.at[pl.ds(eo, C)], si)
        pltpu.sync_copy(dst_hbm.at[pl.ds(eo, C)], di)
        pltpu.async_copy(y_hbm.at[si], yr, sy)
        pltpu.async_copy(rc_hbm.at[pl.ds(eo * H, C * H)], rf.at[pl.ds(0, C * H)], sr)

    def wait(b):
        si, di, yr, rf, sy, sr = bufs[b]
        pltpu.make_async_copy(y_hbm.at[si], yr, sy).wait()
        pltpu.make_async_copy(rc_hbm.at[pl.ds(0, C * H)], rf.at[pl.ds(0, C * H)], sr).wait()

    def compute(b, cc):
        si, di, yr, rf, *_ = bufs[b]

        @plsc.parallel_loop(0, cc, unroll=2)
        def _scale(e):
            bv = plsc.load_gather(rf, [jnp.full((16,), e * H, jnp.int32)])
            for j in range(D // 16):
                msg[e, pl.ds(j * 16, 16)] = yr[e, pl.ds(j * 16, 16)] * bv

        if cc == C:
            pltpu.sync_copy(msg, acc.at[di], add=True)
        else:
            pltpu.sync_copy(msg.at[pl.ds(0, cc)], acc.at[di], add=True)

    issue(0, 0)

    def _body(ci, carry):
        issue(2 * ci + 1, 1)
        wait(0)
        compute(0, C)

        @pl.when(ci < NFULL // 2 - 1)
        def _():
            issue(2 * ci + 2, 0)

        wait(1)
        compute(1, C)
        return carry
    lax.fori_loop(0, NFULL // 2, _body, 0)

    si, di, yr, rf, sy, sr = bufs[0]
    eo = pl.multiple_of(ebase + NFULL * C, 8)
    pltpu.sync_copy(src_hbm.at[pl.ds(eo, REM)], sidx2)
    pltpu.sync_copy(dst_hbm.at[pl.ds(eo, REM)], didx2)
    pltpu.async_copy(y_hbm.at[sidx2], yr.at[pl.ds(0, REM)], sy).wait()
    pltpu.async_copy(rc_hbm.at[pl.ds(eo * H, REM * H)],
                     rf.at[pl.ds(0, REM * H)], sr).wait()

    @plsc.parallel_loop(0, REM, unroll=2)
    def _scale_r(e):
        bv = plsc.load_gather(rf, [jnp.full((16,), e * H, jnp.int32)])
        for j in range(D // 16):
            msg[e, pl.ds(j * 16, 16)] = yr[e, pl.ds(j * 16, 16)] * bv

    pltpu.sync_copy(msg.at[pl.ds(0, REM)], acc.at[didx2], add=True)

    plsc.subcore_barrier()
    _per_subcore_rows(sid, lambda st, cnt: pltpu.sync_copy(
        acc.at[pl.ds(st, cnt)], out_hbm.at[pl.ds(cid * N + st, cnt)]))


def _conv(y, rcf, src, dst, zeros128):
    f = pl.kernel(
        _conv_kernel,
        out_type=jax.ShapeDtypeStruct((NC * N, D), jnp.float32),
        mesh=_MESH,
        compiler_params=pltpu.CompilerParams(needs_layout_passes=False, use_tc_tiling_on_sc=False),
        scratch_types=[
            pltpu.VMEM_SHARED((N, D), jnp.float32),
            pltpu.VMEM((C,), jnp.int32),
            pltpu.VMEM((C,), jnp.int32),
            pltpu.VMEM((C,), jnp.int32),
            pltpu.VMEM((C,), jnp.int32),
            pltpu.VMEM((REM,), jnp.int32),
            pltpu.VMEM((REM,), jnp.int32),
            pltpu.VMEM((C, D), jnp.float32),
            pltpu.VMEM((C * H + 8,), jnp.float32),
            pltpu.VMEM((C, D), jnp.float32),
            pltpu.VMEM((C * H + 8,), jnp.float32),
            pltpu.VMEM((C, D), jnp.float32),
            pltpu.SemaphoreType.DMA,
            pltpu.SemaphoreType.DMA,
            pltpu.SemaphoreType.DMA,
            pltpu.SemaphoreType.DMA,
        ],
    )
    return f(y, rcf, src, dst, zeros128)


def kernel(node_feats, edge_feats, edge_index, edge_attr, Wq, Wk, Wv, Wo,
           R1, b1, R2, gamma, Wself, Wconv, Rc1, bc1, Rc2):
    src = edge_index[0].astype(jnp.int32)
    dst = edge_index[1].astype(jnp.int32)

    ra0, ra1, rc8 = _radials(edge_feats, edge_attr, R1, b1, R2, Rc1, bc1, Rc2)
    radfs = (ra0.reshape(E * H), ra1.reshape(E * H))
    rcf = rc8.reshape(E * H)

    zeros80 = jnp.zeros((N, ACCW), jnp.float32)
    zeros128 = jnp.zeros((N, D), jnp.float32)

    x = node_feats
    for l in range(LAYERS):
        q, k, v = _qkv(x, Wq[l].reshape(D, D), Wk[l].reshape(D, D),
                       Wv[l].reshape(D, DV))
        acc = _edge_attn(q, k, v, radfs[l], src, dst, zeros80)
        x = _epilogue(acc.reshape(NC, N, ACCW), Wo[l], gamma[l])

    xs, y = _finalpre(x, Wself, Wconv)
    ca = _conv(y, rcf, src, dst, zeros128)
    return _finaladd(xs, ca.reshape(NC, N, D))


# speedup vs baseline: 59.2524x; 1.1902x over previous
import functools

import jax
import jax.numpy as jnp
from jax import lax
from jax.experimental import pallas as pl
from jax.experimental.pallas import tpu as pltpu
from jax.experimental.pallas import tpu_sc as plsc

N, E, D, H = 10000, 320000, 128, 8
DE = 4
DV = D // 2
DK = D // H
DVH = DV // H
RH = 32
LAYERS = 2

NC, NS = 2, 16
NW = NC * NS
EPW = E // NW
C = 96
NFULL = EPW // C
REM = EPW - NFULL * C
CV = 64
NFULLV = EPW // CV
REMV = EPW - NFULLV * CV
ACCW = 80
RPT = 632
RPT_LAST = N - (NS - 1) * RPT

BN = 1000
BE = 6400


def _qkv_body(x_ref, wq_ref, wk_ref, wv_ref, q_ref, k_ref, v_ref):
    x = x_ref[...]
    q_ref[...] = (jnp.dot(x, wq_ref[...], preferred_element_type=jnp.float32)
                  * 0.25).astype(jnp.bfloat16)
    k_ref[...] = jnp.dot(x, wk_ref[...],
                         preferred_element_type=jnp.float32).astype(jnp.bfloat16)
    v_ref[...] = jnp.dot(x, wv_ref[...], preferred_element_type=jnp.float32)


def _qkv(x, wq, wk, wv):
    return pl.pallas_call(
        _qkv_body,
        grid=(N // BN,),
        in_specs=[
            pl.BlockSpec((BN, D), lambda i: (i, 0)),
            pl.BlockSpec((D, D), lambda i: (0, 0)),
            pl.BlockSpec((D, D), lambda i: (0, 0)),
            pl.BlockSpec((D, DV), lambda i: (0, 0)),
        ],
        out_specs=[
            pl.BlockSpec((BN, D), lambda i: (i, 0)),
            pl.BlockSpec((BN, D), lambda i: (i, 0)),
            pl.BlockSpec((BN, DV), lambda i: (i, 0)),
        ],
        out_shape=[
            jax.ShapeDtypeStruct((N, D), jnp.bfloat16),
            jax.ShapeDtypeStruct((N, D), jnp.bfloat16),
            jax.ShapeDtypeStruct((N, DV), jnp.float32),
        ],
    )(x, wq, wk, wv)


def _radial_body(efe_ref, ea_ref, r1a_ref, r2a_ref, r1b_ref, r2b_ref,
                 rc1_ref, rc2_ref, b1a_ref, b1b_ref, bc1_ref,
                 ra_ref, rb_ref, rc_ref):
    fe = efe_ref[...]
    ea = ea_ref[...]
    dist = jnp.sqrt(jnp.sum(ea * ea, axis=1, keepdims=True))
    ef = jnp.concatenate([fe, dist], axis=1)

    ha = jnp.maximum(jnp.dot(ef, r1a_ref[...], preferred_element_type=jnp.float32)
                     + b1a_ref[...], 0.0)
    ra_ref[...] = jnp.dot(ha, r2a_ref[...], preferred_element_type=jnp.float32)
    hb = jnp.maximum(jnp.dot(ef, r1b_ref[...], preferred_element_type=jnp.float32)
                     + b1b_ref[...], 0.0)
    rb_ref[...] = jnp.dot(hb, r2b_ref[...], preferred_element_type=jnp.float32)
    hc = jnp.maximum(jnp.dot(ef, rc1_ref[...], preferred_element_type=jnp.float32)
                     + bc1_ref[...], 0.0)
    rcv = jnp.dot(hc, rc2_ref[...], preferred_element_type=jnp.float32)
    rc_ref[...] = rcv * jnp.ones((1, H), jnp.float32)


def _radials(edge_feats, edge_attr, R1, b1, R2, Rc1, bc1, Rc2):
    full = lambda shape: pl.BlockSpec(shape, lambda i: (0, 0))
    return pl.pallas_call(
        _radial_body,
        grid=(E // BE,),
        in_specs=[
            pl.BlockSpec((BE, DE), lambda i: (i, 0)),
            pl.BlockSpec((BE, 3), lambda i: (i, 0)),
            full((DE + 1, RH)), full((RH, H)),
            full((DE + 1, RH)), full((RH, H)),
            full((DE + 1, RH)), full((RH, 1)),
            full((1, RH)), full((1, RH)), full((1, RH)),
        ],
        out_specs=[
            pl.BlockSpec((BE, H), lambda i: (i, 0)),
            pl.BlockSpec((BE, H), lambda i: (i, 0)),
            pl.BlockSpec((BE, H), lambda i: (i, 0)),
        ],
        out_shape=[
            jax.ShapeDtypeStruct((E, H), jnp.float32),
            jax.ShapeDtypeStruct((E, H), jnp.float32),
            jax.ShapeDtypeStruct((E, H), jnp.float32),
        ],
    )(edge_feats, edge_attr, R1[0], R2[0], R1[1], R2[1], Rc1, Rc2,
      b1[0].reshape(1, RH), b1[1].reshape(1, RH), bc1.reshape(1, RH))


def _epilogue_body(acc_ref, wo_ref, g_ref, x_ref):
    a = acc_ref[0] + acc_ref[1]
    aggv = a[:, 0:DV]
    denom = a[:, DV:DV + H]
    r8 = lax.broadcasted_iota(jnp.int32, (H, DV), 0)
    c64 = lax.broadcasted_iota(jnp.int32, (H, DV), 1)
    sel = (c64 // DVH == r8).astype(jnp.float32)
    scale = jnp.dot(1.0 / (denom + 1e-9), sel, preferred_element_type=jnp.float32)
    x = jnp.dot(aggv * scale, wo_ref[...], preferred_element_type=jnp.float32)
    mu = jnp.mean(x, axis=1, keepdims=True)
    var = jnp.mean((x - mu) * (x - mu), axis=1, keepdims=True)
    x_ref[...] = (x - mu) / jnp.sqrt(var + 1e-5) * g_ref[...]


def _epilogue(acc, wo, gamma):
    return pl.pallas_call(
        _epilogue_body,
        grid=(N // BN,),
        in_specs=[
            pl.BlockSpec((NC, BN, ACCW), lambda i: (0, i, 0)),
            pl.BlockSpec((DV, D), lambda i: (0, 0)),
            pl.BlockSpec((1, D), lambda i: (0, 0)),
        ],
        out_specs=pl.BlockSpec((BN, D), lambda i: (i, 0)),
        out_shape=jax.ShapeDtypeStruct((N, D), jnp.float32),
    )(acc, wo, gamma.reshape(1, D))


def _finalpre_body(x_ref, ws_ref, wc_ref, xs_ref, y_ref):
    x = x_ref[...]
    xs_ref[...] = jnp.dot(x, ws_ref[...], preferred_element_type=jnp.float32)
    y_ref[...] = jnp.dot(x, wc_ref[...], preferred_element_type=jnp.float32)


def _finalpre(x, wself, wconv):
    return pl.pallas_call(
        _finalpre_body,
        grid=(N // BN,),
        in_specs=[
            pl.BlockSpec((BN, D), lambda i: (i, 0)),
            pl.BlockSpec((D, D), lambda i: (0, 0)),
            pl.BlockSpec((D, D), lambda i: (0, 0)),
        ],
        out_specs=[
            pl.BlockSpec((BN, D), lambda i: (i, 0)),
            pl.BlockSpec((BN, D), lambda i: (i, 0)),
        ],
        out_shape=[
            jax.ShapeDtypeStruct((N, D), jnp.float32),
            jax.ShapeDtypeStruct((N, D), jnp.float32),
        ],
    )(x, wself, wconv)


def _finaladd_body(xs_ref, ca_ref, o_ref):
    o_ref[...] = xs_ref[...] + ca_ref[0] + ca_ref[1]


def _finaladd(xs, ca):
    return pl.pallas_call(
        _finaladd_body,
        grid=(N // BN,),
        in_specs=[
            pl.BlockSpec((BN, D), lambda i: (i, 0)),
            pl.BlockSpec((NC, BN, D), lambda i: (0, i, 0)),
        ],
        out_specs=pl.BlockSpec((BN, D), lambda i: (i, 0)),
        out_shape=jax.ShapeDtypeStruct((N, D), jnp.float32),
    )(xs, ca)


_MESH = plsc.VectorSubcoreMesh(core_axis_name="c", subcore_axis_name="s",
                               num_cores=NC, num_subcores=NS)


def _per_subcore_rows(sid, fn):
    @pl.when(sid < NS - 1)
    def _():
        fn(sid * RPT, RPT)

    @pl.when(sid == NS - 1)
    def _():
        fn((NS - 1) * RPT, RPT_LAST)


def _edge_attn_kernel(q_hbm, k_hbm, v_hbm, rad_hbm, src_hbm, dst_hbm, z_hbm,
                      out_hbm, acc,
                      sidx0, didx0, sidx1, didx1, sidx2, didx2,
                      dscat0, dscat1,
                      qrow0, krow0, vrow0, radf0,
                      qrow1, krow1, vrow1, radf1,
                      logit, pbuf, pv0, pv1,
                      sq0, sk0, sv0, sr0, sq1, sk1, sv1, sr1, ssc0, ssc1):
    cid = lax.axis_index("c")
    sid = lax.axis_index("s")
    ebase = (sid * NC + cid) * EPW

    lane = lax.broadcasted_iota(jnp.int32, (16,), 0)
    lane_lo = lane < 8
    lo_f = jnp.where(lane_lo, 1.0, 0.0).astype(jnp.float32)
    lane15 = lane == 15
    pbase = jnp.where(lane_lo, 0, 1)

    _per_subcore_rows(sid, lambda st, cnt: pltpu.sync_copy(
        z_hbm.at[pl.ds(st, cnt)], acc.at[pl.ds(st, cnt)]))

    pbuf[pl.ds(C * H - 8, 16)] = jnp.zeros((16,), jnp.float32)

    plsc.subcore_barrier()

    bufs = ((sidx0, didx0, dscat0, qrow0, krow0, vrow0, radf0, pv0,
             sq0, sk0, sv0, sr0, ssc0),
            (sidx1, didx1, dscat1, qrow1, krow1, vrow1, radf1, pv1,
             sq1, sk1, sv1, sr1, ssc1))

    def issue(g, b):
        si, di, dsc, qr, kr, vr, rf, pv, sq, sk, sv, sr, ssc = bufs[b]
        eo = pl.multiple_of(ebase + g * C, 8)
        pltpu.sync_copy(src_hbm.at[pl.ds(eo, C)], si)
        pltpu.sync_copy(dst_hbm.at[pl.ds(eo, C)], di)
        pltpu.async_copy(q_hbm.at[di], qr, sq)
        pltpu.async_copy(k_hbm.at[si], kr, sk)
        pltpu.async_copy(v_hbm.at[si], vr, sv)
        pltpu.async_copy(rad_hbm.at[pl.ds(eo * H, C * H)], rf, sr)

    def wait(b):
        si, di, dsc, qr, kr, vr, rf, pv, sq, sk, sv, sr, ssc = bufs[b]
        pltpu.make_async_copy(q_hbm.at[di], qr, sq).wait()
        pltpu.make_async_copy(k_hbm.at[si], kr, sk).wait()
        pltpu.make_async_copy(v_hbm.at[si], vr, sv).wait()
        pltpu.make_async_copy(rad_hbm.at[pl.ds(0, C * H)], rf, sr).wait()

    def wait_scatter(b):
        si, di, dsc, qr, kr, vr, rf, pv, sq, sk, sv, sr, ssc = bufs[b]
        pltpu.make_async_copy(pv, acc.at[dsc], ssc).wait()

    def dots(qr, kr, rf, cc):
        @plsc.parallel_loop(0, cc // 2, unroll=2)
        def _pair(i):
            for t in range(2):
                e = 2 * i + t
                for g in range(H // 2):
                    qb = qr[e, pl.ds(g * 32, 32)]
                    kb = kr[e, pl.ds(g * 32, 32)]
                    pa, pb = plsc.unpack(qb * kb,
                                         format=plsc.PackFormat.INTERLEAVED)
                    for h, pr in ((2 * g, pa), (2 * g + 1, pb)):
                        s = plsc.cumsum(pr)
                        plsc.store_scatter(
                            logit, [jnp.full((16,), e * H + h, jnp.int32)],
                            s, mask=lane15)
            lv = logit[pl.ds(i * 16, 16)]
            rv = rf[pl.ds(i * 16, 16)]
            pbuf[pl.ds(i * 16, 16)] = jnp.exp(lv * rv)

    def pvfill(vr, pv, cc):
        @plsc.parallel_loop(0, cc, unroll=2)
        def _pv(e):
            p16 = pbuf[pl.ds(e * H, 16)]
            pv[e, pl.ds(DV, 16)] = p16 * lo_f
            for j in range(DV // 16):
                vv = vr[e, pl.ds(j * 16, 16)]
                pj = plsc.load_gather(
                    pbuf, [jnp.full((16,), e * H + 2 * j, jnp.int32) + pbase])
                pv[e, pl.ds(j * 16, 16)] = vv * pj

    def compute(b, ci):
        si, di, dsc, qr, kr, vr, rf, pv, sq, sk, sv, sr, ssc = bufs[b]
        dots(qr, kr, rf, C)

        @pl.when(ci > 0)
        def _():
            wait_scatter(b)

        pvfill(vr, pv, C)
        for i in range(C // 16):
            dsc[pl.ds(i * 16, 16)] = di[pl.ds(i * 16, 16)]
        pltpu.async_copy(pv, acc.at[dsc], ssc, add=True)

    issue(0, 0)

    def _body(ci, carry):
        issue(2 * ci + 1, 1)
        wait(0)
        compute(0, ci)

        @pl.when(ci < NFULL // 2 - 1)
        def _():
            issue(2 * ci + 2, 0)

        wait(1)
        compute(1, ci)
        return carry
    lax.fori_loop(0, NFULL // 2, _body, 0)
    wait_scatter(0)
    wait_scatter(1)

    si, di, dsc, qr, kr, vr, rf, pv, sq, sk, sv, sr, ssc = bufs[0]
    eo = pl.multiple_of(ebase + NFULL * C, 8)
    pltpu.sync_copy(src_hbm.at[pl.ds(eo, REM)], sidx2)
    pltpu.sync_copy(dst_hbm.at[pl.ds(eo, REM)], didx2)
    pltpu.async_copy(q_hbm.at[didx2], qr.at[pl.ds(0, REM)], sq).wait()
    pltpu.async_copy(k_hbm.at[sidx2], kr.at[pl.ds(0, REM)], sk).wait()
    pltpu.async_copy(v_hbm.at[sidx2], vr.at[pl.ds(0, REM)], sv).wait()
    pltpu.async_copy(rad_hbm.at[pl.ds(eo * H, REM * H)],
                     rf.at[pl.ds(0, REM * H)], sr).wait()
    dots(qr, kr, rf, REM)
    pvfill(vr, pv, REM)
    pltpu.sync_copy(pv.at[pl.ds(0, REM)], acc.at[didx2], add=True)

    plsc.subcore_barrier()
    _per_subcore_rows(sid, lambda st, cnt: pltpu.sync_copy(
        acc.at[pl.ds(st, cnt)], out_hbm.at[pl.ds(cid * N + st, cnt)]))


def _edge_attn(q, k, v, radf, src, dst, zeros80):
    f = pl.kernel(
        _edge_attn_kernel,
        out_type=jax.ShapeDtypeStruct((NC * N, ACCW), jnp.float32),
        mesh=_MESH,
        compiler_params=pltpu.CompilerParams(needs_layout_passes=False, use_tc_tiling_on_sc=False),
        scratch_types=[
            pltpu.VMEM_SHARED((N, ACCW), jnp.float32),
            pltpu.VMEM((C,), jnp.int32),
            pltpu.VMEM((C,), jnp.int32),
            pltpu.VMEM((C,), jnp.int32),
            pltpu.VMEM((C,), jnp.int32),
            pltpu.VMEM((REM,), jnp.int32),
            pltpu.VMEM((REM,), jnp.int32),
            pltpu.VMEM((C,), jnp.int32),
            pltpu.VMEM((C,), jnp.int32),
            pltpu.VMEM((C, D), jnp.bfloat16),
            pltpu.VMEM((C, D), jnp.bfloat16),
            pltpu.VMEM((C, DV), jnp.float32),
            pltpu.VMEM((C * H,), jnp.float32),
            pltpu.VMEM((C, D), jnp.bfloat16),
            pltpu.VMEM((C, D), jnp.bfloat16),
            pltpu.VMEM((C, DV), jnp.float32),
            pltpu.VMEM((C * H,), jnp.float32),
            pltpu.VMEM((C * H + 8,), jnp.float32),
            pltpu.VMEM((C * H + 8,), jnp.float32),
            pltpu.VMEM((C, ACCW), jnp.float32),
            pltpu.VMEM((C, ACCW), jnp.float32),
            pltpu.SemaphoreType.DMA,
            pltpu.SemaphoreType.DMA,
            pltpu.SemaphoreType.DMA,
            pltpu.SemaphoreType.DMA,
            pltpu.SemaphoreType.DMA,
            pltpu.SemaphoreType.DMA,
            pltpu.SemaphoreType.DMA,
            pltpu.SemaphoreType.DMA,
            pltpu.SemaphoreType.DMA,
            pltpu.SemaphoreType.DMA,
        ],
    )
    return f(q, k, v, radf, src, dst, zeros80)


def _conv_kernel(y_hbm, rc_hbm, src_hbm, dst_hbm, z_hbm, out_hbm,
                 acc, sidx0, didx0, sidx1, didx1, sidx2, didx2,
                 dscat0, dscat1, yrow0, rcf0, yrow1, rcf1, msg0, msg1,
                 sy0, sr0, sy1, sr1, ssc0, ssc1):
    cid = lax.axis_index("c")
    sid = lax.axis_index("s")
    ebase = (sid * NC + cid) * EPW

    _per_subcore_rows(sid, lambda st, cnt: pltpu.sync_copy(
        z_hbm.at[pl.ds(st, cnt)], acc.at[pl.ds(st, cnt)]))
    plsc.subcore_barrier()

    bufs = ((sidx0, didx0, dscat0, yrow0, rcf0, msg0, sy0, sr0, ssc0),
            (sidx1, didx1, dscat1, yrow1, rcf1, msg1, sy1, sr1, ssc1))

    def issue(g, b):
        si, di, dsc, yr, rf, msg, sy, sr, ssc = bufs[b]
        eo = pl.multiple_of(ebase + g * CV, 8)
        pltpu.sync_copy(src_hbm.at[pl.ds(eo, CV)], si)
        pltpu.sync_copy(dst_hbm.at[pl.ds(eo, CV)], di)
        pltpu.async_copy(y_hbm.at[si], yr, sy)
        pltpu.async_copy(rc_hbm.at[pl.ds(eo * H, CV * H)],
                         rf.at[pl.ds(0, CV * H)], sr)

    def wait(b):
        si, di, dsc, yr, rf, msg, sy, sr, ssc = bufs[b]
        pltpu.make_async_copy(y_hbm.at[si], yr, sy).wait()
        pltpu.make_async_copy(rc_hbm.at[pl.ds(0, CV * H)],
                              rf.at[pl.ds(0, CV * H)], sr).wait()

    def wait_scatter(b):
        si, di, dsc, yr, rf, msg, sy, sr, ssc = bufs[b]
        pltpu.make_async_copy(msg, acc.at[dsc], ssc).wait()

    def scale(yr, rf, msg, cc):
        @plsc.parallel_loop(0, cc, unroll=2)
        def _scale(e):
            bv = plsc.load_gather(rf, [jnp.full((16,), e * H, jnp.int32)])
            for j in range(D // 16):
                msg[e, pl.ds(j * 16, 16)] = yr[e, pl.ds(j * 16, 16)] * bv

    def compute(b, ci):
        si, di, dsc, yr, rf, msg, sy, sr, ssc = bufs[b]

        @pl.when(ci > 0)
        def _():
            wait_scatter(b)

        scale(yr, rf, msg, CV)
        for i in range(CV // 16):
            dsc[pl.ds(i * 16, 16)] = di[pl.ds(i * 16, 16)]
        pltpu.async_copy(msg, acc.at[dsc], ssc, add=True)

    issue(0, 0)

    def _body(ci, carry):
        issue(2 * ci + 1, 1)
        wait(0)
        compute(0, ci)

        @pl.when(ci < NFULLV // 2 - 1)
        def _():
            issue(2 * ci + 2, 0)

        wait(1)
        compute(1, ci)
        return carry
    lax.fori_loop(0, NFULLV // 2, _body, 0)
    wait_scatter(0)
    wait_scatter(1)

    si, di, dsc, yr, rf, msg, sy, sr, ssc = bufs[0]
    eo = pl.multiple_of(ebase + NFULLV * CV, 8)
    pltpu.sync_copy(src_hbm.at[pl.ds(eo, REMV)], sidx2)
    pltpu.sync_copy(dst_hbm.at[pl.ds(eo, REMV)], didx2)
    pltpu.async_copy(y_hbm.at[sidx2], yr.at[pl.ds(0, REMV)], sy).wait()
    pltpu.async_copy(rc_hbm.at[pl.ds(eo * H, REMV * H)],
                     rf.at[pl.ds(0, REMV * H)], sr).wait()
    scale(yr, rf, msg, REMV)
    pltpu.sync_copy(msg.at[pl.ds(0, REMV)], acc.at[didx2], add=True)

    plsc.subcore_barrier()
    _per_subcore_rows(sid, lambda st, cnt: pltpu.sync_copy(
        acc.at[pl.ds(st, cnt)], out_hbm.at[pl.ds(cid * N + st, cnt)]))


def _conv(y, rcf, src, dst, zeros128):
    f = pl.kernel(
        _conv_kernel,
        out_type=jax.ShapeDtypeStruct((NC * N, D), jnp.float32),
        mesh=_MESH,
        compiler_params=pltpu.CompilerParams(needs_layout_passes=False, use_tc_tiling_on_sc=False),
        scratch_types=[
            pltpu.VMEM_SHARED((N, D), jnp.float32),
            pltpu.VMEM((CV,), jnp.int32),
            pltpu.VMEM((CV,), jnp.int32),
            pltpu.VMEM((CV,), jnp.int32),
            pltpu.VMEM((CV,), jnp.int32),
            pltpu.VMEM((REMV,), jnp.int32),
            pltpu.VMEM((REMV,), jnp.int32),
            pltpu.VMEM((CV,), jnp.int32),
            pltpu.VMEM((CV,), jnp.int32),
            pltpu.VMEM((CV, D), jnp.float32),
            pltpu.VMEM((CV * H + 8,), jnp.float32),
            pltpu.VMEM((CV, D), jnp.float32),
            pltpu.VMEM((CV * H + 8,), jnp.float32),
            pltpu.VMEM((CV, D), jnp.float32),
            pltpu.VMEM((CV, D), jnp.float32),
            pltpu.SemaphoreType.DMA,
            pltpu.SemaphoreType.DMA,
            pltpu.SemaphoreType.DMA,
            pltpu.SemaphoreType.DMA,
            pltpu.SemaphoreType.DMA,
            pltpu.SemaphoreType.DMA,
        ],
    )
    return f(y, rcf, src, dst, zeros128)


def kernel(node_feats, edge_feats, edge_index, edge_attr, Wq, Wk, Wv, Wo,
           R1, b1, R2, gamma, Wself, Wconv, Rc1, bc1, Rc2):
    src = edge_index[0].astype(jnp.int32)
    dst = edge_index[1].astype(jnp.int32)

    ra0, ra1, rc8 = _radials(edge_feats, edge_attr, R1, b1, R2, Rc1, bc1, Rc2)
    radfs = (ra0.reshape(E * H), ra1.reshape(E * H))
    rcf = rc8.reshape(E * H)

    zeros80 = jnp.zeros((N, ACCW), jnp.float32)
    zeros128 = jnp.zeros((N, D), jnp.float32)

    qkp = jnp.asarray([(2 * g + s) * DK + t
                       for g in range(H // 2) for t in range(DK) for s in (0, 1)],
                      dtype=jnp.int32)

    x = node_feats
    for l in range(LAYERS):
        q, k, v = _qkv(x, Wq[l].reshape(D, D)[:, qkp], Wk[l].reshape(D, D)[:, qkp],
                       Wv[l].reshape(D, DV))
        acc = _edge_attn(q, k, v, radfs[l], src, dst, zeros80)
        x = _epilogue(acc.reshape(NC, N, ACCW), Wo[l], gamma[l])

    xs, y = _finalpre(x, Wself, Wconv)
    ca = _conv(y, rcf, src, dst, zeros128)
    return _finaladd(xs, ca.reshape(NC, N, D))


# speedup vs baseline: 76.2818x; 1.2874x over previous
import functools

import jax
import jax.numpy as jnp
from jax import lax
from jax.experimental import pallas as pl
from jax.experimental.pallas import tpu as pltpu
from jax.experimental.pallas import tpu_sc as plsc

N, E, D, H = 10000, 320000, 128, 8
DE = 4
DV = D // 2
DK = D // H
DVH = DV // H
RH = 32
LAYERS = 2

NC, NS = 2, 16
NW = NC * NS
EPW = E // NW
C = 96
NFULL = EPW // C
REM = EPW - NFULL * C
CV = 64
NFULLV = EPW // CV
REMV = EPW - NFULLV * CV
ACCW = 80
RPT = 632
RPT_LAST = N - (NS - 1) * RPT

BN = 1000
BE = 6400


def _qkv_body(x_ref, wq_ref, wk_ref, wv_ref, q_ref, k_ref, v_ref):
    x = x_ref[...]
    q_ref[...] = (jnp.dot(x, wq_ref[...], preferred_element_type=jnp.float32)
                  * 0.25).astype(jnp.bfloat16)
    k_ref[...] = jnp.dot(x, wk_ref[...],
                         preferred_element_type=jnp.float32).astype(jnp.bfloat16)
    v_ref[...] = jnp.dot(x, wv_ref[...], preferred_element_type=jnp.float32)


def _qkv(x, wq, wk, wv):
    return pl.pallas_call(
        _qkv_body,
        grid=(N // BN,),
        in_specs=[
            pl.BlockSpec((BN, D), lambda i: (i, 0)),
            pl.BlockSpec((D, D), lambda i: (0, 0)),
            pl.BlockSpec((D, D), lambda i: (0, 0)),
            pl.BlockSpec((D, DV), lambda i: (0, 0)),
        ],
        out_specs=[
            pl.BlockSpec((BN, D), lambda i: (i, 0)),
            pl.BlockSpec((BN, D), lambda i: (i, 0)),
            pl.BlockSpec((BN, DV), lambda i: (i, 0)),
        ],
        out_shape=[
            jax.ShapeDtypeStruct((N, D), jnp.bfloat16),
            jax.ShapeDtypeStruct((N, D), jnp.bfloat16),
            jax.ShapeDtypeStruct((N, DV), jnp.float32),
        ],
    )(x, wq, wk, wv)


def _radial_body(fet_ref, eat_ref, r1a_ref, r2a_ref, r1b_ref, r2b_ref,
                 rc1_ref, rc2_ref, b1a_ref, b1b_ref, bc1_ref,
                 ra_ref, rb_ref, rc_ref):
    fet = fet_ref[...]
    eat = eat_ref[...]
    dist = jnp.sqrt(jnp.sum(eat * eat, axis=0, keepdims=True))
    eft = jnp.concatenate([fet, dist], axis=0)

    ha = jnp.maximum(jnp.dot(r1a_ref[...], eft, preferred_element_type=jnp.float32)
                     + b1a_ref[...], 0.0)
    ra_ref[...] = jnp.dot(r2a_ref[...], ha, preferred_element_type=jnp.float32)
    hb = jnp.maximum(jnp.dot(r1b_ref[...], eft, preferred_element_type=jnp.float32)
                     + b1b_ref[...], 0.0)
    rb_ref[...] = jnp.dot(r2b_ref[...], hb, preferred_element_type=jnp.float32)
    hc = jnp.maximum(jnp.dot(rc1_ref[...], eft, preferred_element_type=jnp.float32)
                     + bc1_ref[...], 0.0)
    rc_ref[...] = jnp.dot(rc2_ref[...], hc, preferred_element_type=jnp.float32)


def _radials(edge_feats_t, edge_attr_t, R1, b1, R2, Rc1, bc1, Rc2):
    full = lambda shape: pl.BlockSpec(shape, lambda i: (0, 0))
    return pl.pallas_call(
        _radial_body,
        grid=(E // BE,),
        in_specs=[
            pl.BlockSpec((DE, BE), lambda i: (0, i)),
            pl.BlockSpec((3, BE), lambda i: (0, i)),
            full((RH, DE + 1)), full((H, RH)),
            full((RH, DE + 1)), full((H, RH)),
            full((RH, DE + 1)), full((1, RH)),
            full((RH, 1)), full((RH, 1)), full((RH, 1)),
        ],
        out_specs=[
            pl.BlockSpec((H, BE), lambda i: (0, i)),
            pl.BlockSpec((H, BE), lambda i: (0, i)),
            pl.BlockSpec((1, BE), lambda i: (0, i)),
        ],
        out_shape=[
            jax.ShapeDtypeStruct((H, E), jnp.float32),
            jax.ShapeDtypeStruct((H, E), jnp.float32),
            jax.ShapeDtypeStruct((1, E), jnp.float32),
        ],
    )(edge_feats_t, edge_attr_t,
      R1[0].T, R2[0].T, R1[1].T, R2[1].T, Rc1.T, Rc2.T,
      b1[0].reshape(RH, 1), b1[1].reshape(RH, 1), bc1.reshape(RH, 1))


def _epilogue_body(acc_ref, wo_ref, g_ref, x_ref):
    a = acc_ref[0] + acc_ref[1]
    aggv = a[:, 0:DV]
    denom = a[:, DV:DV + H]
    r8 = lax.broadcasted_iota(jnp.int32, (H, DV), 0)
    c64 = lax.broadcasted_iota(jnp.int32, (H, DV), 1)
    sel = (c64 // DVH == r8).astype(jnp.float32)
    scale = jnp.dot(1.0 / (denom + 1e-9), sel, preferred_element_type=jnp.float32)
    x = jnp.dot(aggv * scale, wo_ref[...], preferred_element_type=jnp.float32)
    mu = jnp.mean(x, axis=1, keepdims=True)
    var = jnp.mean((x - mu) * (x - mu), axis=1, keepdims=True)
    x_ref[...] = (x - mu) / jnp.sqrt(var + 1e-5) * g_ref[...]


def _epilogue(acc, wo, gamma):
    return pl.pallas_call(
        _epilogue_body,
        grid=(N // BN,),
        in_specs=[
            pl.BlockSpec((NC, BN, ACCW), lambda i: (0, i, 0)),
            pl.BlockSpec((DV, D), lambda i: (0, 0)),
            pl.BlockSpec((1, D), lambda i: (0, 0)),
        ],
        out_specs=pl.BlockSpec((BN, D), lambda i: (i, 0)),
        out_shape=jax.ShapeDtypeStruct((N, D), jnp.float32),
    )(acc, wo, gamma.reshape(1, D))


def _finalpre_body(x_ref, ws_ref, wc_ref, xs_ref, y_ref):
    x = x_ref[...]
    xs_ref[...] = jnp.dot(x, ws_ref[...], preferred_element_type=jnp.float32)
    y_ref[...] = jnp.dot(x, wc_ref[...], preferred_element_type=jnp.float32)


def _finalpre(x, wself, wconv):
    return pl.pallas_call(
        _finalpre_body,
        grid=(N // BN,),
        in_specs=[
            pl.BlockSpec((BN, D), lambda i: (i, 0)),
            pl.BlockSpec((D, D), lambda i: (0, 0)),
            pl.BlockSpec((D, D), lambda i: (0, 0)),
        ],
        out_specs=[
            pl.BlockSpec((BN, D), lambda i: (i, 0)),
            pl.BlockSpec((BN, D), lambda i: (i, 0)),
        ],
        out_shape=[
            jax.ShapeDtypeStruct((N, D), jnp.float32),
            jax.ShapeDtypeStruct((N, D), jnp.float32),
        ],
    )(x, wself, wconv)


def _finaladd_body(xs_ref, ca_ref, o_ref):
    o_ref[...] = xs_ref[...] + ca_ref[0] + ca_ref[1]


def _finaladd(xs, ca):
    return pl.pallas_call(
        _finaladd_body,
        grid=(N // BN,),
        in_specs=[
            pl.BlockSpec((BN, D), lambda i: (i, 0)),
            pl.BlockSpec((NC, BN, D), lambda i: (0, i, 0)),
        ],
        out_specs=pl.BlockSpec((BN, D), lambda i: (i, 0)),
        out_shape=jax.ShapeDtypeStruct((N, D), jnp.float32),
    )(xs, ca)


_MESH = plsc.VectorSubcoreMesh(core_axis_name="c", subcore_axis_name="s",
                               num_cores=NC, num_subcores=NS)


def _per_subcore_rows(sid, fn):
    @pl.when(sid < NS - 1)
    def _():
        fn(sid * RPT, RPT)

    @pl.when(sid == NS - 1)
    def _():
        fn((NS - 1) * RPT, RPT_LAST)


def _edge_attn_kernel(q_hbm, k_hbm, v_hbm, rad_hbm, src_hbm, dst_hbm, z_hbm,
                      out_hbm, acc,
                      sidx0, didx0, sidx1, didx1, sidx2, didx2,
                      dscat0, dscat1,
                      qrow0, krow0, vrow0, radf0,
                      qrow1, krow1, vrow1, radf1,
                      logit, pbuf, pv0, pv1,
                      sq0, sk0, sv0, sr0, sq1, sk1, sv1, sr1, ssc0, ssc1):
    cid = lax.axis_index("c")
    sid = lax.axis_index("s")
    ebase = (sid * NC + cid) * EPW

    lane = lax.broadcasted_iota(jnp.int32, (16,), 0)
    lane_lo = lane < 8
    lo_f = jnp.where(lane_lo, 1.0, 0.0).astype(jnp.float32)
    lane15 = lane == 15
    pbase = jnp.where(lane_lo, 0, 1)

    _per_subcore_rows(sid, lambda st, cnt: pltpu.sync_copy(
        z_hbm.at[pl.ds(st, cnt)], acc.at[pl.ds(st, cnt)]))

    pbuf[pl.ds(C * H - 8, 16)] = jnp.zeros((16,), jnp.float32)

    plsc.subcore_barrier()

    bufs = ((sidx0, didx0, dscat0, qrow0, krow0, vrow0, radf0, pv0,
             sq0, sk0, sv0, sr0, ssc0),
            (sidx1, didx1, dscat1, qrow1, krow1, vrow1, radf1, pv1,
             sq1, sk1, sv1, sr1, ssc1))

    def issue(g, b):
        si, di, dsc, qr, kr, vr, rf, pv, sq, sk, sv, sr, ssc = bufs[b]
        eo = pl.multiple_of(ebase + g * C, 8)
        pltpu.sync_copy(src_hbm.at[pl.ds(eo, C)], si)
        pltpu.sync_copy(dst_hbm.at[pl.ds(eo, C)], di)
        pltpu.async_copy(q_hbm.at[di], qr, sq)
        pltpu.async_copy(k_hbm.at[si], kr, sk)
        pltpu.async_copy(v_hbm.at[si], vr, sv)
        pltpu.async_copy(rad_hbm.at[pl.ds(eo * H, C * H)], rf, sr)

    def wait(b):
        si, di, dsc, qr, kr, vr, rf, pv, sq, sk, sv, sr, ssc = bufs[b]
        pltpu.make_async_copy(q_hbm.at[di], qr, sq).wait()
        pltpu.make_async_copy(k_hbm.at[si], kr, sk).wait()
        pltpu.make_async_copy(v_hbm.at[si], vr, sv).wait()
        pltpu.make_async_copy(rad_hbm.at[pl.ds(0, C * H)], rf, sr).wait()

    def wait_scatter(b):
        si, di, dsc, qr, kr, vr, rf, pv, sq, sk, sv, sr, ssc = bufs[b]
        pltpu.make_async_copy(pv, acc.at[dsc], ssc).wait()

    def dots(qr, kr, rf, cc):
        @plsc.parallel_loop(0, cc // 2, unroll=2)
        def _pair(i):
            for t in range(2):
                e = 2 * i + t
                for g in range(H // 2):
                    qb = qr[e, pl.ds(g * 32, 32)]
                    kb = kr[e, pl.ds(g * 32, 32)]
                    pa, pb = plsc.unpack(qb * kb,
                                         format=plsc.PackFormat.INTERLEAVED)
                    for h, pr in ((2 * g, pa), (2 * g + 1, pb)):
                        s = plsc.cumsum(pr)
                        plsc.store_scatter(
                            logit, [jnp.full((16,), e * H + h, jnp.int32)],
                            s, mask=lane15)
            lv = logit[pl.ds(i * 16, 16)]
            rv = rf[pl.ds(i * 16, 16)]
            pbuf[pl.ds(i * 16, 16)] = jnp.exp(lv * rv)

    def pvfill(vr, pv, cc):
        @plsc.parallel_loop(0, cc, unroll=2)
        def _pv(e):
            p16 = pbuf[pl.ds(e * H, 16)]
            pv[e, pl.ds(DV, 16)] = p16 * lo_f
            for j in range(DV // 16):
                vv = vr[e, pl.ds(j * 16, 16)]
                pj = plsc.load_gather(
                    pbuf, [jnp.full((16,), e * H + 2 * j, jnp.int32) + pbase])
                pv[e, pl.ds(j * 16, 16)] = vv * pj

    def compute(b, ci):
        si, di, dsc, qr, kr, vr, rf, pv, sq, sk, sv, sr, ssc = bufs[b]
        dots(qr, kr, rf, C)

        @pl.when(ci > 0)
        def _():
            wait_scatter(b)

        pvfill(vr, pv, C)
        for i in range(C // 16):
            dsc[pl.ds(i * 16, 16)] = di[pl.ds(i * 16, 16)]
        pltpu.async_copy(pv, acc.at[dsc], ssc, add=True)

    issue(0, 0)

    def _body(ci, carry):
        issue(2 * ci + 1, 1)
        wait(0)
        compute(0, ci)

        @pl.when(ci < NFULL // 2 - 1)
        def _():
            issue(2 * ci + 2, 0)

        wait(1)
        compute(1, ci)
        return carry
    lax.fori_loop(0, NFULL // 2, _body, 0)
    wait_scatter(0)
    wait_scatter(1)

    si, di, dsc, qr, kr, vr, rf, pv, sq, sk, sv, sr, ssc = bufs[0]
    eo = pl.multiple_of(ebase + NFULL * C, 8)
    pltpu.sync_copy(src_hbm.at[pl.ds(eo, REM)], sidx2)
    pltpu.sync_copy(dst_hbm.at[pl.ds(eo, REM)], didx2)
    pltpu.async_copy(q_hbm.at[didx2], qr.at[pl.ds(0, REM)], sq).wait()
    pltpu.async_copy(k_hbm.at[sidx2], kr.at[pl.ds(0, REM)], sk).wait()
    pltpu.async_copy(v_hbm.at[sidx2], vr.at[pl.ds(0, REM)], sv).wait()
    pltpu.async_copy(rad_hbm.at[pl.ds(eo * H, REM * H)],
                     rf.at[pl.ds(0, REM * H)], sr).wait()
    dots(qr, kr, rf, REM)
    pvfill(vr, pv, REM)
    pltpu.sync_copy(pv.at[pl.ds(0, REM)], acc.at[didx2], add=True)

    plsc.subcore_barrier()
    _per_subcore_rows(sid, lambda st, cnt: pltpu.sync_copy(
        acc.at[pl.ds(st, cnt)], out_hbm.at[pl.ds(cid * N + st, cnt)]))


def _edge_attn(q, k, v, radf, src, dst, zeros80):
    f = pl.kernel(
        _edge_attn_kernel,
        out_type=jax.ShapeDtypeStruct((NC * N, ACCW), jnp.float32),
        mesh=_MESH,
        compiler_params=pltpu.CompilerParams(needs_layout_passes=False, use_tc_tiling_on_sc=False),
        scratch_types=[
            pltpu.VMEM_SHARED((N, ACCW), jnp.float32),
            pltpu.VMEM((C,), jnp.int32),
            pltpu.VMEM((C,), jnp.int32),
            pltpu.VMEM((C,), jnp.int32),
            pltpu.VMEM((C,), jnp.int32),
            pltpu.VMEM((REM,), jnp.int32),
            pltpu.VMEM((REM,), jnp.int32),
            pltpu.VMEM((C,), jnp.int32),
            pltpu.VMEM((C,), jnp.int32),
            pltpu.VMEM((C, D), jnp.bfloat16),
            pltpu.VMEM((C, D), jnp.bfloat16),
            pltpu.VMEM((C, DV), jnp.float32),
            pltpu.VMEM((C * H,), jnp.float32),
            pltpu.VMEM((C, D), jnp.bfloat16),
            pltpu.VMEM((C, D), jnp.bfloat16),
            pltpu.VMEM((C, DV), jnp.float32),
            pltpu.VMEM((C * H,), jnp.float32),
            pltpu.VMEM((C * H + 8,), jnp.float32),
            pltpu.VMEM((C * H + 8,), jnp.float32),
            pltpu.VMEM((C, ACCW), jnp.float32),
            pltpu.VMEM((C, ACCW), jnp.float32),
            pltpu.SemaphoreType.DMA,
            pltpu.SemaphoreType.DMA,
            pltpu.SemaphoreType.DMA,
            pltpu.SemaphoreType.DMA,
            pltpu.SemaphoreType.DMA,
            pltpu.SemaphoreType.DMA,
            pltpu.SemaphoreType.DMA,
            pltpu.SemaphoreType.DMA,
            pltpu.SemaphoreType.DMA,
            pltpu.SemaphoreType.DMA,
        ],
    )
    return f(q, k, v, radf, src, dst, zeros80)


def _conv_kernel(y_hbm, rc_hbm, src_hbm, dst_hbm, z_hbm, out_hbm,
                 acc, sidx0, didx0, sidx1, didx1, sidx2, didx2,
                 dscat0, dscat1, yrow0, rcf0, yrow1, rcf1, msg0, msg1,
                 sy0, sr0, sy1, sr1, ssc0, ssc1):
    cid = lax.axis_index("c")
    sid = lax.axis_index("s")
    ebase = (sid * NC + cid) * EPW

    _per_subcore_rows(sid, lambda st, cnt: pltpu.sync_copy(
        z_hbm.at[pl.ds(st, cnt)], acc.at[pl.ds(st, cnt)]))
    plsc.subcore_barrier()

    bufs = ((sidx0, didx0, dscat0, yrow0, rcf0, msg0, sy0, sr0, ssc0),
            (sidx1, didx1, dscat1, yrow1, rcf1, msg1, sy1, sr1, ssc1))

    def issue(g, b):
        si, di, dsc, yr, rf, msg, sy, sr, ssc = bufs[b]
        eo = pl.multiple_of(ebase + g * CV, 8)
        pltpu.sync_copy(src_hbm.at[pl.ds(eo, CV)], si)
        pltpu.sync_copy(dst_hbm.at[pl.ds(eo, CV)], di)
        pltpu.async_copy(y_hbm.at[si], yr, sy)
        pltpu.async_copy(rc_hbm.at[pl.ds(eo, CV)], rf, sr)

    def wait(b):
        si, di, dsc, yr, rf, msg, sy, sr, ssc = bufs[b]
        pltpu.make_async_copy(y_hbm.at[si], yr, sy).wait()
        pltpu.make_async_copy(rc_hbm.at[pl.ds(0, CV)], rf, sr).wait()

    def wait_scatter(b):
        si, di, dsc, yr, rf, msg, sy, sr, ssc = bufs[b]
        pltpu.make_async_copy(msg, acc.at[dsc], ssc).wait()

    def scale(yr, rf, msg, cc):
        @plsc.parallel_loop(0, cc, unroll=2)
        def _scale(e):
            bv = plsc.load_gather(rf, [jnp.full((16,), e, jnp.int32)])
            for j in range(D // 16):
                msg[e, pl.ds(j * 16, 16)] = yr[e, pl.ds(j * 16, 16)] * bv

    def compute(b, ci):
        si, di, dsc, yr, rf, msg, sy, sr, ssc = bufs[b]

        @pl.when(ci > 0)
        def _():
            wait_scatter(b)

        scale(yr, rf, msg, CV)
        for i in range(CV // 16):
            dsc[pl.ds(i * 16, 16)] = di[pl.ds(i * 16, 16)]
        pltpu.async_copy(msg, acc.at[dsc], ssc, add=True)

    issue(0, 0)

    def _body(ci, carry):
        issue(2 * ci + 1, 1)
        wait(0)
        compute(0, ci)

        @pl.when(ci < NFULLV // 2 - 1)
        def _():
            issue(2 * ci + 2, 0)

        wait(1)
        compute(1, ci)
        return carry
    lax.fori_loop(0, NFULLV // 2, _body, 0)
    wait_scatter(0)
    wait_scatter(1)

    si, di, dsc, yr, rf, msg, sy, sr, ssc = bufs[0]
    eo = pl.multiple_of(ebase + NFULLV * CV, 8)
    pltpu.sync_copy(src_hbm.at[pl.ds(eo, REMV)], sidx2)
    pltpu.sync_copy(dst_hbm.at[pl.ds(eo, REMV)], didx2)
    pltpu.async_copy(y_hbm.at[sidx2], yr.at[pl.ds(0, REMV)], sy).wait()
    pltpu.async_copy(rc_hbm.at[pl.ds(eo, REMV)], rf.at[pl.ds(0, REMV)], sr).wait()
    scale(yr, rf, msg, REMV)
    pltpu.sync_copy(msg.at[pl.ds(0, REMV)], acc.at[didx2], add=True)

    plsc.subcore_barrier()
    _per_subcore_rows(sid, lambda st, cnt: pltpu.sync_copy(
        acc.at[pl.ds(st, cnt)], out_hbm.at[pl.ds(cid * N + st, cnt)]))


def _conv(y, rcf, src, dst, zeros128):
    f = pl.kernel(
        _conv_kernel,
        out_type=jax.ShapeDtypeStruct((NC * N, D), jnp.float32),
        mesh=_MESH,
        compiler_params=pltpu.CompilerParams(needs_layout_passes=False, use_tc_tiling_on_sc=False),
        scratch_types=[
            pltpu.VMEM_SHARED((N, D), jnp.float32),
            pltpu.VMEM((CV,), jnp.int32),
            pltpu.VMEM((CV,), jnp.int32),
            pltpu.VMEM((CV,), jnp.int32),
            pltpu.VMEM((CV,), jnp.int32),
            pltpu.VMEM((REMV,), jnp.int32),
            pltpu.VMEM((REMV,), jnp.int32),
            pltpu.VMEM((CV,), jnp.int32),
            pltpu.VMEM((CV,), jnp.int32),
            pltpu.VMEM((CV, D), jnp.float32),
            pltpu.VMEM((CV,), jnp.float32),
            pltpu.VMEM((CV, D), jnp.float32),
            pltpu.VMEM((CV,), jnp.float32),
            pltpu.VMEM((CV, D), jnp.float32),
            pltpu.VMEM((CV, D), jnp.float32),
            pltpu.SemaphoreType.DMA,
            pltpu.SemaphoreType.DMA,
            pltpu.SemaphoreType.DMA,
            pltpu.SemaphoreType.DMA,
            pltpu.SemaphoreType.DMA,
            pltpu.SemaphoreType.DMA,
        ],
    )
    return f(y, rcf, src, dst, zeros128)


def kernel(node_feats, edge_feats, edge_index, edge_attr, Wq, Wk, Wv, Wo,
           R1, b1, R2, gamma, Wself, Wconv, Rc1, bc1, Rc2):
    src = edge_index[0].astype(jnp.int32)
    dst = edge_index[1].astype(jnp.int32)

    ra0t, ra1t, rct = _radials(edge_feats.T, edge_attr.T, R1, b1, R2,
                               Rc1, bc1, Rc2)
    radfs = (ra0t.T.reshape(E * H), ra1t.T.reshape(E * H))
    rcf = rct.reshape(E)

    zeros80 = jnp.zeros((N, ACCW), jnp.float32)
    zeros128 = jnp.zeros((N, D), jnp.float32)

    qkp = jnp.asarray([(2 * g + s) * DK + t
                       for g in range(H // 2) for t in range(DK) for s in (0, 1)],
                      dtype=jnp.int32)

    x = node_feats
    for l in range(LAYERS):
        q, k, v = _qkv(x, Wq[l].reshape(D, D)[:, qkp], Wk[l].reshape(D, D)[:, qkp],
                       Wv[l].reshape(D, DV))
        acc = _edge_attn(q, k, v, radfs[l], src, dst, zeros80)
        x = _epilogue(acc.reshape(NC, N, ACCW), Wo[l], gamma[l])

    xs, y = _finalpre(x, Wself, Wconv)
    ca = _conv(y, rcf, src, dst, zeros128)
    return _finaladd(xs, ca.reshape(NC, N, D))


# speedup vs baseline: 76.2963x; 1.0002x over previous
import functools

import jax
import jax.numpy as jnp
from jax import lax
from jax.experimental import pallas as pl
from jax.experimental.pallas import tpu as pltpu
from jax.experimental.pallas import tpu_sc as plsc

N, E, D, H = 10000, 320000, 128, 8
DE = 4
DV = D // 2
DK = D // H
DVH = DV // H
RH = 32
LAYERS = 2

NC, NS = 2, 16
NW = NC * NS
EPW = E // NW
C = 96
NFULL = EPW // C
REM = EPW - NFULL * C
CV = 64
NFULLV = EPW // CV
REMV = EPW - NFULLV * CV
ACCW = 80
RPT = 632
RPT_LAST = N - (NS - 1) * RPT

BN = 1000
BE = 6400


def _qkv_body(x_ref, wq_ref, wk_ref, wv_ref, q_ref, k_ref, v_ref):
    x = x_ref[...]
    q_ref[...] = (jnp.dot(x, wq_ref[...], preferred_element_type=jnp.float32)
                  * 0.25).astype(jnp.bfloat16)
    k_ref[...] = jnp.dot(x, wk_ref[...],
                         preferred_element_type=jnp.float32).astype(jnp.bfloat16)
    v_ref[...] = jnp.dot(x, wv_ref[...], preferred_element_type=jnp.float32)


def _qkv(x, wq, wk, wv):
    return pl.pallas_call(
        _qkv_body,
        grid=(N // BN,),
        in_specs=[
            pl.BlockSpec((BN, D), lambda i: (i, 0)),
            pl.BlockSpec((D, D), lambda i: (0, 0)),
            pl.BlockSpec((D, D), lambda i: (0, 0)),
            pl.BlockSpec((D, DV), lambda i: (0, 0)),
        ],
        out_specs=[
            pl.BlockSpec((BN, D), lambda i: (i, 0)),
            pl.BlockSpec((BN, D), lambda i: (i, 0)),
            pl.BlockSpec((BN, DV), lambda i: (i, 0)),
        ],
        out_shape=[
            jax.ShapeDtypeStruct((N, D), jnp.bfloat16),
            jax.ShapeDtypeStruct((N, D), jnp.bfloat16),
            jax.ShapeDtypeStruct((N, DV), jnp.float32),
        ],
    )(x, wq, wk, wv)


def _radial_body(fet_ref, eat_ref, r1a_ref, r2a_ref, r1b_ref, r2b_ref,
                 rc1_ref, rc2_ref, b1a_ref, b1b_ref, bc1_ref,
                 ra_ref, rb_ref, rc_ref):
    fet = fet_ref[...]
    eat = eat_ref[...]
    dist = jnp.sqrt(jnp.sum(eat * eat, axis=0, keepdims=True))
    eft = jnp.concatenate([fet, dist], axis=0)

    ha = jnp.maximum(jnp.dot(r1a_ref[...], eft, preferred_element_type=jnp.float32)
                     + b1a_ref[...], 0.0)
    ra_ref[...] = jnp.dot(r2a_ref[...], ha, preferred_element_type=jnp.float32)
    hb = jnp.maximum(jnp.dot(r1b_ref[...], eft, preferred_element_type=jnp.float32)
                     + b1b_ref[...], 0.0)
    rb_ref[...] = jnp.dot(r2b_ref[...], hb, preferred_element_type=jnp.float32)
    hc = jnp.maximum(jnp.dot(rc1_ref[...], eft, preferred_element_type=jnp.float32)
                     + bc1_ref[...], 0.0)
    rc_ref[...] = jnp.dot(rc2_ref[...], hc, preferred_element_type=jnp.float32)


def _radials(edge_feats_t, edge_attr_t, R1, b1, R2, Rc1, bc1, Rc2):
    full = lambda shape: pl.BlockSpec(shape, lambda i: (0, 0))
    return pl.pallas_call(
        _radial_body,
        grid=(E // BE,),
        in_specs=[
            pl.BlockSpec((DE, BE), lambda i: (0, i)),
            pl.BlockSpec((3, BE), lambda i: (0, i)),
            full((RH, DE + 1)), full((H, RH)),
            full((RH, DE + 1)), full((H, RH)),
            full((RH, DE + 1)), full((1, RH)),
            full((RH, 1)), full((RH, 1)), full((RH, 1)),
        ],
        out_specs=[
            pl.BlockSpec((H, BE), lambda i: (0, i)),
            pl.BlockSpec((H, BE), lambda i: (0, i)),
            pl.BlockSpec((1, BE), lambda i: (0, i)),
        ],
        out_shape=[
            jax.ShapeDtypeStruct((H, E), jnp.float32),
            jax.ShapeDtypeStruct((H, E), jnp.float32),
            jax.ShapeDtypeStruct((1, E), jnp.float32),
        ],
    )(edge_feats_t, edge_attr_t,
      R1[0].T, R2[0].T, R1[1].T, R2[1].T, Rc1.T, Rc2.T,
      b1[0].reshape(RH, 1), b1[1].reshape(RH, 1), bc1.reshape(RH, 1))


def _epilogue_body(acc_ref, wo_ref, g_ref, x_ref):
    a = acc_ref[0] + acc_ref[1]
    aggv = a[:, 0:DV]
    denom = a[:, DV:DV + H]
    r8 = lax.broadcasted_iota(jnp.int32, (H, DV), 0)
    c64 = lax.broadcasted_iota(jnp.int32, (H, DV), 1)
    sel = (c64 // DVH == r8).astype(jnp.float32)
    scale = jnp.dot(1.0 / (denom + 1e-9), sel, preferred_element_type=jnp.float32)
    x = jnp.dot(aggv * scale, wo_ref[...], preferred_element_type=jnp.float32)
    mu = jnp.mean(x, axis=1, keepdims=True)
    var = jnp.mean((x - mu) * (x - mu), axis=1, keepdims=True)
    x_ref[...] = (x - mu) / jnp.sqrt(var + 1e-5) * g_ref[...]


def _epilogue(acc, wo, gamma):
    return pl.pallas_call(
        _epilogue_body,
        grid=(N // BN,),
        in_specs=[
            pl.BlockSpec((NC, BN, ACCW), lambda i: (0, i, 0)),
            pl.BlockSpec((DV, D), lambda i: (0, 0)),
            pl.BlockSpec((1, D), lambda i: (0, 0)),
        ],
        out_specs=pl.BlockSpec((BN, D), lambda i: (i, 0)),
        out_shape=jax.ShapeDtypeStruct((N, D), jnp.float32),
    )(acc, wo, gamma.reshape(1, D))


def _finalpre_body(x_ref, ws_ref, wc_ref, xs_ref, y_ref):
    x = x_ref[...]
    xs_ref[...] = jnp.dot(x, ws_ref[...], preferred_element_type=jnp.float32)
    y_ref[...] = jnp.dot(x, wc_ref[...], preferred_element_type=jnp.float32)


def _finalpre(x, wself, wconv):
    return pl.pallas_call(
        _finalpre_body,
        grid=(N // BN,),
        in_specs=[
            pl.BlockSpec((BN, D), lambda i: (i, 0)),
            pl.BlockSpec((D, D), lambda i: (0, 0)),
            pl.BlockSpec((D, D), lambda i: (0, 0)),
        ],
        out_specs=[
            pl.BlockSpec((BN, D), lambda i: (i, 0)),
            pl.BlockSpec((BN, D), lambda i: (i, 0)),
        ],
        out_shape=[
            jax.ShapeDtypeStruct((N, D), jnp.float32),
            jax.ShapeDtypeStruct((N, D), jnp.float32),
        ],
    )(x, wself, wconv)


def _finaladd_body(xs_ref, ca_ref, o_ref):
    o_ref[...] = xs_ref[...] + ca_ref[0] + ca_ref[1]


def _finaladd(xs, ca):
    return pl.pallas_call(
        _finaladd_body,
        grid=(N // BN,),
        in_specs=[
            pl.BlockSpec((BN, D), lambda i: (i, 0)),
            pl.BlockSpec((NC, BN, D), lambda i: (0, i, 0)),
        ],
        out_specs=pl.BlockSpec((BN, D), lambda i: (i, 0)),
        out_shape=jax.ShapeDtypeStruct((N, D), jnp.float32),
    )(xs, ca)


_MESH = plsc.VectorSubcoreMesh(core_axis_name="c", subcore_axis_name="s",
                               num_cores=NC, num_subcores=NS)


def _per_subcore_rows(sid, fn):
    @pl.when(sid < NS - 1)
    def _():
        fn(sid * RPT, RPT)

    @pl.when(sid == NS - 1)
    def _():
        fn((NS - 1) * RPT, RPT_LAST)


def _edge_attn_kernel(q_hbm, k_hbm, v_hbm, rad_hbm, src_hbm, dst_hbm, z_hbm,
                      out_hbm, acc,
                      sidx0, didx0, sidx1, didx1, sidx2, didx2,
                      dscat0, dscat1,
                      qrow0, krow0, vrow0, radf0,
                      qrow1, krow1, vrow1, radf1,
                      logit, pbuf, pv0, pv1,
                      sq0, sk0, sv0, sr0, sq1, sk1, sv1, sr1, ssc0, ssc1):
    cid = lax.axis_index("c")
    sid = lax.axis_index("s")
    ebase = (sid * NC + cid) * EPW

    lane = lax.broadcasted_iota(jnp.int32, (16,), 0)
    lane_lo = lane < 8
    lo_f = jnp.where(lane_lo, 1.0, 0.0).astype(jnp.float32)
    lane15 = lane == 15
    pbase = jnp.where(lane_lo, 0, 1)

    _per_subcore_rows(sid, lambda st, cnt: pltpu.sync_copy(
        z_hbm.at[pl.ds(st, cnt)], acc.at[pl.ds(st, cnt)]))

    pbuf[pl.ds(C * H - 8, 16)] = jnp.zeros((16,), jnp.float32)

    plsc.subcore_barrier()

    bufs = ((sidx0, didx0, dscat0, qrow0, krow0, vrow0, radf0, pv0,
             sq0, sk0, sv0, sr0, ssc0),
            (sidx1, didx1, dscat1, qrow1, krow1, vrow1, radf1, pv1,
             sq1, sk1, sv1, sr1, ssc1))

    def issue(g, b):
        si, di, dsc, qr, kr, vr, rf, pv, sq, sk, sv, sr, ssc = bufs[b]
        eo = pl.multiple_of(ebase + g * C, 8)
        pltpu.sync_copy(src_hbm.at[pl.ds(eo, C)], si)
        pltpu.sync_copy(dst_hbm.at[pl.ds(eo, C)], di)
        pltpu.async_copy(q_hbm.at[di], qr, sq)
        pltpu.async_copy(k_hbm.at[si], kr, sk)
        pltpu.async_copy(v_hbm.at[si], vr, sv)
        pltpu.async_copy(rad_hbm.at[pl.ds(eo * H, C * H)], rf, sr)

    def wait(b):
        si, di, dsc, qr, kr, vr, rf, pv, sq, sk, sv, sr, ssc = bufs[b]
        pltpu.make_async_copy(q_hbm.at[di], qr, sq).wait()
        pltpu.make_async_copy(k_hbm.at[si], kr, sk).wait()
        pltpu.make_async_copy(v_hbm.at[si], vr, sv).wait()
        pltpu.make_async_copy(rad_hbm.at[pl.ds(0, C * H)], rf, sr).wait()

    def wait_scatter(b):
        si, di, dsc, qr, kr, vr, rf, pv, sq, sk, sv, sr, ssc = bufs[b]
        pltpu.make_async_copy(pv, acc.at[dsc], ssc).wait()

    def dots(qr, kr, rf, cc):
        @plsc.parallel_loop(0, cc // 2, unroll=3)
        def _pair(i):
            for t in range(2):
                e = 2 * i + t
                for g in range(H // 2):
                    qb = qr[e, pl.ds(g * 32, 32)]
                    kb = kr[e, pl.ds(g * 32, 32)]
                    pa, pb = plsc.unpack(qb * kb,
                                         format=plsc.PackFormat.INTERLEAVED)
                    for h, pr in ((2 * g, pa), (2 * g + 1, pb)):
                        s = plsc.cumsum(pr)
                        plsc.store_scatter(
                            logit, [jnp.full((16,), e * H + h, jnp.int32)],
                            s, mask=lane15)
            lv = logit[pl.ds(i * 16, 16)]
            rv = rf[pl.ds(i * 16, 16)]
            pbuf[pl.ds(i * 16, 16)] = jnp.exp(lv * rv)

    def pvfill(vr, pv, cc):
        @plsc.parallel_loop(0, cc, unroll=4)
        def _pv(e):
            p16 = pbuf[pl.ds(e * H, 16)]
            pv[e, pl.ds(DV, 16)] = p16 * lo_f
            for j in range(DV // 16):
                vv = vr[e, pl.ds(j * 16, 16)]
                pj = plsc.load_gather(
                    pbuf, [jnp.full((16,), e * H + 2 * j, jnp.int32) + pbase])
                pv[e, pl.ds(j * 16, 16)] = vv * pj

    def compute(b, ci):
        si, di, dsc, qr, kr, vr, rf, pv, sq, sk, sv, sr, ssc = bufs[b]
        dots(qr, kr, rf, C)

        @pl.when(ci > 0)
        def _():
            wait_scatter(b)

        pvfill(vr, pv, C)
        for i in range(C // 16):
            dsc[pl.ds(i * 16, 16)] = di[pl.ds(i * 16, 16)]
        pltpu.async_copy(pv, acc.at[dsc], ssc, add=True)

    issue(0, 0)

    def _body(ci, carry):
        issue(2 * ci + 1, 1)
        wait(0)
        compute(0, ci)

        @pl.when(ci < NFULL // 2 - 1)
        def _():
            issue(2 * ci + 2, 0)

        wait(1)
        compute(1, ci)
        return carry
    lax.fori_loop(0, NFULL // 2, _body, 0)
    wait_scatter(0)
    wait_scatter(1)

    si, di, dsc, qr, kr, vr, rf, pv, sq, sk, sv, sr, ssc = bufs[0]
    eo = pl.multiple_of(ebase + NFULL * C, 8)
    pltpu.sync_copy(src_hbm.at[pl.ds(eo, REM)], sidx2)
    pltpu.sync_copy(dst_hbm.at[pl.ds(eo, REM)], didx2)
    pltpu.async_copy(q_hbm.at[didx2], qr.at[pl.ds(0, REM)], sq).wait()
    pltpu.async_copy(k_hbm.at[sidx2], kr.at[pl.ds(0, REM)], sk).wait()
    pltpu.async_copy(v_hbm.at[sidx2], vr.at[pl.ds(0, REM)], sv).wait()
    pltpu.async_copy(rad_hbm.at[pl.ds(eo * H, REM * H)],
                     rf.at[pl.ds(0, REM * H)], sr).wait()
    dots(qr, kr, rf, REM)
    pvfill(vr, pv, REM)
    pltpu.sync_copy(pv.at[pl.ds(0, REM)], acc.at[didx2], add=True)

    plsc.subcore_barrier()
    _per_subcore_rows(sid, lambda st, cnt: pltpu.sync_copy(
        acc.at[pl.ds(st, cnt)], out_hbm.at[pl.ds(cid * N + st, cnt)]))


def _edge_attn(q, k, v, radf, src, dst, zeros80):
    f = pl.kernel(
        _edge_attn_kernel,
        out_type=jax.ShapeDtypeStruct((NC * N, ACCW), jnp.float32),
        mesh=_MESH,
        compiler_params=pltpu.CompilerParams(needs_layout_passes=False, use_tc_tiling_on_sc=False),
        scratch_types=[
            pltpu.VMEM_SHARED((N, ACCW), jnp.float32),
            pltpu.VMEM((C,), jnp.int32),
            pltpu.VMEM((C,), jnp.int32),
            pltpu.VMEM((C,), jnp.int32),
            pltpu.VMEM((C,), jnp.int32),
            pltpu.VMEM((REM,), jnp.int32),
            pltpu.VMEM((REM,), jnp.int32),
            pltpu.VMEM((C,), jnp.int32),
            pltpu.VMEM((C,), jnp.int32),
            pltpu.VMEM((C, D), jnp.bfloat16),
            pltpu.VMEM((C, D), jnp.bfloat16),
            pltpu.VMEM((C, DV), jnp.float32),
            pltpu.VMEM((C * H,), jnp.float32),
            pltpu.VMEM((C, D), jnp.bfloat16),
            pltpu.VMEM((C, D), jnp.bfloat16),
            pltpu.VMEM((C, DV), jnp.float32),
            pltpu.VMEM((C * H,), jnp.float32),
            pltpu.VMEM((C * H + 8,), jnp.float32),
            pltpu.VMEM((C * H + 8,), jnp.float32),
            pltpu.VMEM((C, ACCW), jnp.float32),
            pltpu.VMEM((C, ACCW), jnp.float32),
            pltpu.SemaphoreType.DMA,
            pltpu.SemaphoreType.DMA,
            pltpu.SemaphoreType.DMA,
            pltpu.SemaphoreType.DMA,
            pltpu.SemaphoreType.DMA,
            pltpu.SemaphoreType.DMA,
            pltpu.SemaphoreType.DMA,
            pltpu.SemaphoreType.DMA,
            pltpu.SemaphoreType.DMA,
            pltpu.SemaphoreType.DMA,
        ],
    )
    return f(q, k, v, radf, src, dst, zeros80)


def _conv_kernel(y_hbm, rc_hbm, src_hbm, dst_hbm, z_hbm, out_hbm,
                 acc, sidx0, didx0, sidx1, didx1, sidx2, didx2,
                 dscat0, dscat1, yrow0, rcf0, yrow1, rcf1, msg0, msg1,
                 sy0, sr0, sy1, sr1, ssc0, ssc1):
    cid = lax.axis_index("c")
    sid = lax.axis_index("s")
    ebase = (sid * NC + cid) * EPW

    _per_subcore_rows(sid, lambda st, cnt: pltpu.sync_copy(
        z_hbm.at[pl.ds(st, cnt)], acc.at[pl.ds(st, cnt)]))
    plsc.subcore_barrier()

    bufs = ((sidx0, didx0, dscat0, yrow0, rcf0, msg0, sy0, sr0, ssc0),
            (sidx1, didx1, dscat1, yrow1, rcf1, msg1, sy1, sr1, ssc1))

    def issue(g, b):
        si, di, dsc, yr, rf, msg, sy, sr, ssc = bufs[b]
        eo = pl.multiple_of(ebase + g * CV, 8)
        pltpu.sync_copy(src_hbm.at[pl.ds(eo, CV)], si)
        pltpu.sync_copy(dst_hbm.at[pl.ds(eo, CV)], di)
        pltpu.async_copy(y_hbm.at[si], yr, sy)
        pltpu.async_copy(rc_hbm.at[pl.ds(eo, CV)], rf, sr)

    def wait(b):
        si, di, dsc, yr, rf, msg, sy, sr, ssc = bufs[b]
        pltpu.make_async_copy(y_hbm.at[si], yr, sy).wait()
        pltpu.make_async_copy(rc_hbm.at[pl.ds(0, CV)], rf, sr).wait()

    def wait_scatter(b):
        si, di, dsc, yr, rf, msg, sy, sr, ssc = bufs[b]
        pltpu.make_async_copy(msg, acc.at[dsc], ssc).wait()

    def scale(yr, rf, msg, cc):
        @plsc.parallel_loop(0, cc, unroll=2)
        def _scale(e):
            bv = plsc.load_gather(rf, [jnp.full((16,), e, jnp.int32)])
            for j in range(D // 16):
                msg[e, pl.ds(j * 16, 16)] = yr[e, pl.ds(j * 16, 16)] * bv

    def compute(b, ci):
        si, di, dsc, yr, rf, msg, sy, sr, ssc = bufs[b]

        @pl.when(ci > 0)
        def _():
            wait_scatter(b)

        scale(yr, rf, msg, CV)
        for i in range(CV // 16):
            dsc[pl.ds(i * 16, 16)] = di[pl.ds(i * 16, 16)]
        pltpu.async_copy(msg, acc.at[dsc], ssc, add=True)

    issue(0, 0)

    def _body(ci, carry):
        issue(2 * ci + 1, 1)
        wait(0)
        compute(0, ci)

        @pl.when(ci < NFULLV // 2 - 1)
        def _():
            issue(2 * ci + 2, 0)

        wait(1)
        compute(1, ci)
        return carry
    lax.fori_loop(0, NFULLV // 2, _body, 0)
    wait_scatter(0)
    wait_scatter(1)

    si, di, dsc, yr, rf, msg, sy, sr, ssc = bufs[0]
    eo = pl.multiple_of(ebase + NFULLV * CV, 8)
    pltpu.sync_copy(src_hbm.at[pl.ds(eo, REMV)], sidx2)
    pltpu.sync_copy(dst_hbm.at[pl.ds(eo, REMV)], didx2)
    pltpu.async_copy(y_hbm.at[sidx2], yr.at[pl.ds(0, REMV)], sy).wait()
    pltpu.async_copy(rc_hbm.at[pl.ds(eo, REMV)], rf.at[pl.ds(0, REMV)], sr).wait()
    scale(yr, rf, msg, REMV)
    pltpu.sync_copy(msg.at[pl.ds(0, REMV)], acc.at[didx2], add=True)

    plsc.subcore_barrier()
    _per_subcore_rows(sid, lambda st, cnt: pltpu.sync_copy(
        acc.at[pl.ds(st, cnt)], out_hbm.at[pl.ds(cid * N + st, cnt)]))


def _conv(y, rcf, src, dst, zeros128):
    f = pl.kernel(
        _conv_kernel,
        out_type=jax.ShapeDtypeStruct((NC * N, D), jnp.float32),
        mesh=_MESH,
        compiler_params=pltpu.CompilerParams(needs_layout_passes=False, use_tc_tiling_on_sc=False),
        scratch_types=[
            pltpu.VMEM_SHARED((N, D), jnp.float32),
            pltpu.VMEM((CV,), jnp.int32),
            pltpu.VMEM((CV,), jnp.int32),
            pltpu.VMEM((CV,), jnp.int32),
            pltpu.VMEM((CV,), jnp.int32),
            pltpu.VMEM((REMV,), jnp.int32),
            pltpu.VMEM((REMV,), jnp.int32),
            pltpu.VMEM((CV,), jnp.int32),
            pltpu.VMEM((CV,), jnp.int32),
            pltpu.VMEM((CV, D), jnp.float32),
            pltpu.VMEM((CV,), jnp.float32),
            pltpu.VMEM((CV, D), jnp.float32),
            pltpu.VMEM((CV,), jnp.float32),
            pltpu.VMEM((CV, D), jnp.float32),
            pltpu.VMEM((CV, D), jnp.float32),
            pltpu.SemaphoreType.DMA,
            pltpu.SemaphoreType.DMA,
            pltpu.SemaphoreType.DMA,
            pltpu.SemaphoreType.DMA,
            pltpu.SemaphoreType.DMA,
            pltpu.SemaphoreType.DMA,
        ],
    )
    return f(y, rcf, src, dst, zeros128)


def kernel(node_feats, edge_feats, edge_index, edge_attr, Wq, Wk, Wv, Wo,
           R1, b1, R2, gamma, Wself, Wconv, Rc1, bc1, Rc2):
    src = edge_index[0].astype(jnp.int32)
    dst = edge_index[1].astype(jnp.int32)

    ra0t, ra1t, rct = _radials(edge_feats.T, edge_attr.T, R1, b1, R2,
                               Rc1, bc1, Rc2)
    radfs = (ra0t.T.reshape(E * H), ra1t.T.reshape(E * H))
    rcf = rct.reshape(E)

    zeros80 = jnp.zeros((N, ACCW), jnp.float32)
    zeros128 = jnp.zeros((N, D), jnp.float32)

    qkp = jnp.asarray([(2 * g + s) * DK + t
                       for g in range(H // 2) for t in range(DK) for s in (0, 1)],
                      dtype=jnp.int32)

    x = node_feats
    for l in range(LAYERS):
        q, k, v = _qkv(x, Wq[l].reshape(D, D)[:, qkp], Wk[l].reshape(D, D)[:, qkp],
                       Wv[l].reshape(D, DV))
        acc = _edge_attn(q, k, v, radfs[l], src, dst, zeros80)
        x = _epilogue(acc.reshape(NC, N, ACCW), Wo[l], gamma[l])

    xs, y = _finalpre(x, Wself, Wconv)
    ca = _conv(y, rcf, src, dst, zeros128)
    return _finaladd(xs, ca.reshape(NC, N, D))


# speedup vs baseline: 79.3062x; 1.0394x over previous
import functools

import jax
import jax.numpy as jnp
from jax import lax
from jax.experimental import pallas as pl
from jax.experimental.pallas import tpu as pltpu
from jax.experimental.pallas import tpu_sc as plsc

N, E, D, H = 10000, 320000, 128, 8
DE = 4
DV = D // 2
DK = D // H
DVH = DV // H
RH = 32
LAYERS = 2

NC, NS = 2, 16
NW = NC * NS
EPW = E // NW
C = 96
NFULL = EPW // C
REM = EPW - NFULL * C
CV = 64
NFULLV = EPW // CV
REMV = EPW - NFULLV * CV
ACCW = 80
RPT = 632
RPT_LAST = N - (NS - 1) * RPT

BN = 1000
BE = 6400


def _qkv_body(x_ref, wq_ref, wk_ref, wv_ref, q_ref, k_ref, v_ref):
    x = x_ref[...]
    q_ref[...] = (jnp.dot(x, wq_ref[...], preferred_element_type=jnp.float32)
                  * 0.25).astype(jnp.bfloat16)
    k_ref[...] = jnp.dot(x, wk_ref[...],
                         preferred_element_type=jnp.float32).astype(jnp.bfloat16)
    v_ref[...] = jnp.dot(x, wv_ref[...],
                         preferred_element_type=jnp.float32).astype(jnp.bfloat16)


def _qkv(x, wq, wk, wv):
    return pl.pallas_call(
        _qkv_body,
        grid=(N // BN,),
        in_specs=[
            pl.BlockSpec((BN, D), lambda i: (i, 0)),
            pl.BlockSpec((D, D), lambda i: (0, 0)),
            pl.BlockSpec((D, D), lambda i: (0, 0)),
            pl.BlockSpec((D, DV), lambda i: (0, 0)),
        ],
        out_specs=[
            pl.BlockSpec((BN, D), lambda i: (i, 0)),
            pl.BlockSpec((BN, D), lambda i: (i, 0)),
            pl.BlockSpec((BN, DV), lambda i: (i, 0)),
        ],
        out_shape=[
            jax.ShapeDtypeStruct((N, D), jnp.bfloat16),
            jax.ShapeDtypeStruct((N, D), jnp.bfloat16),
            jax.ShapeDtypeStruct((N, DV), jnp.bfloat16),
        ],
    )(x, wq, wk, wv)


def _radial_body(fet_ref, eat_ref, r1a_ref, r2a_ref, r1b_ref, r2b_ref,
                 rc1_ref, rc2_ref, b1a_ref, b1b_ref, bc1_ref,
                 ra_ref, rb_ref, rc_ref):
    fet = fet_ref[...]
    eat = eat_ref[...]
    dist = jnp.sqrt(jnp.sum(eat * eat, axis=0, keepdims=True))
    eft = jnp.concatenate([fet, dist], axis=0)

    ha = jnp.maximum(jnp.dot(r1a_ref[...], eft, preferred_element_type=jnp.float32)
                     + b1a_ref[...], 0.0)
    ra_ref[...] = jnp.dot(r2a_ref[...], ha, preferred_element_type=jnp.float32)
    hb = jnp.maximum(jnp.dot(r1b_ref[...], eft, preferred_element_type=jnp.float32)
                     + b1b_ref[...], 0.0)
    rb_ref[...] = jnp.dot(r2b_ref[...], hb, preferred_element_type=jnp.float32)
    hc = jnp.maximum(jnp.dot(rc1_ref[...], eft, preferred_element_type=jnp.float32)
                     + bc1_ref[...], 0.0)
    rc_ref[...] = jnp.dot(rc2_ref[...], hc, preferred_element_type=jnp.float32)


def _radials(edge_feats_t, edge_attr_t, R1, b1, R2, Rc1, bc1, Rc2):
    full = lambda shape: pl.BlockSpec(shape, lambda i: (0, 0))
    return pl.pallas_call(
        _radial_body,
        grid=(E // BE,),
        in_specs=[
            pl.BlockSpec((DE, BE), lambda i: (0, i)),
            pl.BlockSpec((3, BE), lambda i: (0, i)),
            full((RH, DE + 1)), full((H, RH)),
            full((RH, DE + 1)), full((H, RH)),
            full((RH, DE + 1)), full((1, RH)),
            full((RH, 1)), full((RH, 1)), full((RH, 1)),
        ],
        out_specs=[
            pl.BlockSpec((H, BE), lambda i: (0, i)),
            pl.BlockSpec((H, BE), lambda i: (0, i)),
            pl.BlockSpec((1, BE), lambda i: (0, i)),
        ],
        out_shape=[
            jax.ShapeDtypeStruct((H, E), jnp.float32),
            jax.ShapeDtypeStruct((H, E), jnp.float32),
            jax.ShapeDtypeStruct((1, E), jnp.float32),
        ],
    )(edge_feats_t, edge_attr_t,
      R1[0].T, R2[0].T, R1[1].T, R2[1].T, Rc1.T, Rc2.T,
      b1[0].reshape(RH, 1), b1[1].reshape(RH, 1), bc1.reshape(RH, 1))


def _epilogue_body(acc_ref, wo_ref, g_ref, x_ref):
    a = acc_ref[0] + acc_ref[1]
    aggv = a[:, 0:DV]
    denom = a[:, DV:DV + H]
    r8 = lax.broadcasted_iota(jnp.int32, (H, DV), 0)
    c64 = lax.broadcasted_iota(jnp.int32, (H, DV), 1)
    sel = (c64 // DVH == r8).astype(jnp.float32)
    scale = jnp.dot(1.0 / (denom + 1e-9), sel, preferred_element_type=jnp.float32)
    x = jnp.dot(aggv * scale, wo_ref[...], preferred_element_type=jnp.float32)
    mu = jnp.mean(x, axis=1, keepdims=True)
    var = jnp.mean((x - mu) * (x - mu), axis=1, keepdims=True)
    x_ref[...] = (x - mu) / jnp.sqrt(var + 1e-5) * g_ref[...]


def _epilogue(acc, wo, gamma):
    return pl.pallas_call(
        _epilogue_body,
        grid=(N // BN,),
        in_specs=[
            pl.BlockSpec((NC, BN, ACCW), lambda i: (0, i, 0)),
            pl.BlockSpec((DV, D), lambda i: (0, 0)),
            pl.BlockSpec((1, D), lambda i: (0, 0)),
        ],
        out_specs=pl.BlockSpec((BN, D), lambda i: (i, 0)),
        out_shape=jax.ShapeDtypeStruct((N, D), jnp.float32),
    )(acc, wo, gamma.reshape(1, D))


def _finalpre_body(x_ref, ws_ref, wc_ref, xs_ref, y_ref):
    x = x_ref[...]
    xs_ref[...] = jnp.dot(x, ws_ref[...], preferred_element_type=jnp.float32)
    y_ref[...] = jnp.dot(x, wc_ref[...],
                         preferred_element_type=jnp.float32).astype(jnp.bfloat16)


def _finalpre(x, wself, wconv):
    return pl.pallas_call(
        _finalpre_body,
        grid=(N // BN,),
        in_specs=[
            pl.BlockSpec((BN, D), lambda i: (i, 0)),
            pl.BlockSpec((D, D), lambda i: (0, 0)),
            pl.BlockSpec((D, D), lambda i: (0, 0)),
        ],
        out_specs=[
            pl.BlockSpec((BN, D), lambda i: (i, 0)),
            pl.BlockSpec((BN, D), lambda i: (i, 0)),
        ],
        out_shape=[
            jax.ShapeDtypeStruct((N, D), jnp.float32),
            jax.ShapeDtypeStruct((N, D), jnp.bfloat16),
        ],
    )(x, wself, wconv)


def _finaladd_body(xs_ref, ca_ref, o_ref):
    o_ref[...] = xs_ref[...] + ca_ref[0] + ca_ref[1]


def _finaladd(xs, ca):
    return pl.pallas_call(
        _finaladd_body,
        grid=(N // BN,),
        in_specs=[
            pl.BlockSpec((BN, D), lambda i: (i, 0)),
            pl.BlockSpec((NC, BN, D), lambda i: (0, i, 0)),
        ],
        out_specs=pl.BlockSpec((BN, D), lambda i: (i, 0)),
        out_shape=jax.ShapeDtypeStruct((N, D), jnp.float32),
    )(xs, ca)


_MESH = plsc.VectorSubcoreMesh(core_axis_name="c", subcore_axis_name="s",
                               num_cores=NC, num_subcores=NS)


def _per_subcore_rows(sid, fn):
    @pl.when(sid < NS - 1)
    def _():
        fn(sid * RPT, RPT)

    @pl.when(sid == NS - 1)
    def _():
        fn((NS - 1) * RPT, RPT_LAST)


def _edge_attn_kernel(q_hbm, k_hbm, v_hbm, rad_hbm, src_hbm, dst_hbm, z_hbm,
                      out_hbm, acc,
                      sidx0, didx0, sidx1, didx1, sidx2, didx2,
                      dscat0, dscat1,
                      qrow0, krow0, vrow0, radf0,
                      qrow1, krow1, vrow1, radf1,
                      logit, pbuf, pv0, pv1,
                      sq0, sk0, sv0, sr0, sq1, sk1, sv1, sr1, ssc0, ssc1):
    cid = lax.axis_index("c")
    sid = lax.axis_index("s")
    ebase = (sid * NC + cid) * EPW

    lane = lax.broadcasted_iota(jnp.int32, (16,), 0)
    lane_lo = lane < 8
    lo_f = jnp.where(lane_lo, 1.0, 0.0).astype(jnp.float32)
    lane15 = lane == 15
    pbase = jnp.where(lane_lo, 0, 1)

    _per_subcore_rows(sid, lambda st, cnt: pltpu.sync_copy(
        z_hbm.at[pl.ds(st, cnt)], acc.at[pl.ds(st, cnt)]))

    pbuf[pl.ds(C * H - 8, 16)] = jnp.zeros((16,), jnp.float32)

    plsc.subcore_barrier()

    bufs = ((sidx0, didx0, dscat0, qrow0, krow0, vrow0, radf0, pv0,
             sq0, sk0, sv0, sr0, ssc0),
            (sidx1, didx1, dscat1, qrow1, krow1, vrow1, radf1, pv1,
             sq1, sk1, sv1, sr1, ssc1))

    def issue(g, b):
        si, di, dsc, qr, kr, vr, rf, pv, sq, sk, sv, sr, ssc = bufs[b]
        eo = pl.multiple_of(ebase + g * C, 8)
        pltpu.sync_copy(src_hbm.at[pl.ds(eo, C)], si)
        pltpu.sync_copy(dst_hbm.at[pl.ds(eo, C)], di)
        pltpu.async_copy(q_hbm.at[di], qr, sq)
        pltpu.async_copy(k_hbm.at[si], kr, sk)
        pltpu.async_copy(v_hbm.at[si], vr, sv)
        pltpu.async_copy(rad_hbm.at[pl.ds(eo * H, C * H)], rf, sr)

    def wait(b):
        si, di, dsc, qr, kr, vr, rf, pv, sq, sk, sv, sr, ssc = bufs[b]
        pltpu.make_async_copy(q_hbm.at[di], qr, sq).wait()
        pltpu.make_async_copy(k_hbm.at[si], kr, sk).wait()
        pltpu.make_async_copy(v_hbm.at[si], vr, sv).wait()
        pltpu.make_async_copy(rad_hbm.at[pl.ds(0, C * H)], rf, sr).wait()

    def wait_scatter(b):
        si, di, dsc, qr, kr, vr, rf, pv, sq, sk, sv, sr, ssc = bufs[b]
        pltpu.make_async_copy(pv, acc.at[dsc], ssc).wait()

    def dots(qr, kr, rf, cc):
        @plsc.parallel_loop(0, cc // 2, unroll=3)
        def _pair(i):
            for t in range(2):
                e = 2 * i + t
                for g in range(H // 2):
                    qb = qr[e, pl.ds(g * 32, 32)]
                    kb = kr[e, pl.ds(g * 32, 32)]
                    pa, pb = plsc.unpack(qb * kb,
                                         format=plsc.PackFormat.INTERLEAVED)
                    for h, pr in ((2 * g, pa), (2 * g + 1, pb)):
                        s = plsc.cumsum(pr)
                        plsc.store_scatter(
                            logit, [jnp.full((16,), e * H + h, jnp.int32)],
                            s, mask=lane15)
            lv = logit[pl.ds(i * 16, 16)]
            rv = rf[pl.ds(i * 16, 16)]
            pbuf[pl.ds(i * 16, 16)] = jnp.exp(lv * rv)

    def pvfill(vr, pv, cc):
        pass_holder = None
        @plsc.parallel_loop(0, cc, unroll=4)
        def _pv(e):
            p16 = pbuf[pl.ds(e * H, 16)]
            pv[e, pl.ds(DV, 16)] = p16 * lo_f
            for g in range(DV // 32):
                v32 = vr[e, pl.ds(g * 32, 32)]
                va, vb = plsc.unpack(v32, format=plsc.PackFormat.INTERLEAVED)
                for j, vv in ((2 * g, va), (2 * g + 1, vb)):
                    pj = plsc.load_gather(
                        pbuf, [jnp.full((16,), e * H + 2 * j, jnp.int32) + pbase])
                    pv[e, pl.ds(j * 16, 16)] = vv * pj

    def compute(b, ci):
        si, di, dsc, qr, kr, vr, rf, pv, sq, sk, sv, sr, ssc = bufs[b]
        dots(qr, kr, rf, C)

        @pl.when(ci > 0)
        def _():
            wait_scatter(b)

        pvfill(vr, pv, C)
        for i in range(C // 16):
            dsc[pl.ds(i * 16, 16)] = di[pl.ds(i * 16, 16)]
        pltpu.async_copy(pv, acc.at[dsc], ssc, add=True)

    issue(0, 0)

    def _body(ci, carry):
        issue(2 * ci + 1, 1)
        wait(0)
        compute(0, ci)

        @pl.when(ci < NFULL // 2 - 1)
        def _():
            issue(2 * ci + 2, 0)

        wait(1)
        compute(1, ci)
        return carry
    lax.fori_loop(0, NFULL // 2, _body, 0)
    wait_scatter(0)
    wait_scatter(1)

    si, di, dsc, qr, kr, vr, rf, pv, sq, sk, sv, sr, ssc = bufs[0]
    eo = pl.multiple_of(ebase + NFULL * C, 8)
    pltpu.sync_copy(src_hbm.at[pl.ds(eo, REM)], sidx2)
    pltpu.sync_copy(dst_hbm.at[pl.ds(eo, REM)], didx2)
    pltpu.async_copy(q_hbm.at[didx2], qr.at[pl.ds(0, REM)], sq).wait()
    pltpu.async_copy(k_hbm.at[sidx2], kr.at[pl.ds(0, REM)], sk).wait()
    pltpu.async_copy(v_hbm.at[sidx2], vr.at[pl.ds(0, REM)], sv).wait()
    pltpu.async_copy(rad_hbm.at[pl.ds(eo * H, REM * H)],
                     rf.at[pl.ds(0, REM * H)], sr).wait()
    dots(qr, kr, rf, REM)
    pvfill(vr, pv, REM)
    pltpu.sync_copy(pv.at[pl.ds(0, REM)], acc.at[didx2], add=True)

    plsc.subcore_barrier()
    _per_subcore_rows(sid, lambda st, cnt: pltpu.sync_copy(
        acc.at[pl.ds(st, cnt)], out_hbm.at[pl.ds(cid * N + st, cnt)]))


def _edge_attn(q, k, v, radf, src, dst, zeros80):
    f = pl.kernel(
        _edge_attn_kernel,
        out_type=jax.ShapeDtypeStruct((NC * N, ACCW), jnp.float32),
        mesh=_MESH,
        compiler_params=pltpu.CompilerParams(needs_layout_passes=False, use_tc_tiling_on_sc=False),
        scratch_types=[
            pltpu.VMEM_SHARED((N, ACCW), jnp.float32),
            pltpu.VMEM((C,), jnp.int32),
            pltpu.VMEM((C,), jnp.int32),
            pltpu.VMEM((C,), jnp.int32),
            pltpu.VMEM((C,), jnp.int32),
            pltpu.VMEM((REM,), jnp.int32),
            pltpu.VMEM((REM,), jnp.int32),
            pltpu.VMEM((C,), jnp.int32),
            pltpu.VMEM((C,), jnp.int32),
            pltpu.VMEM((C, D), jnp.bfloat16),
            pltpu.VMEM((C, D), jnp.bfloat16),
            pltpu.VMEM((C, DV), jnp.bfloat16),
            pltpu.VMEM((C * H,), jnp.float32),
            pltpu.VMEM((C, D), jnp.bfloat16),
            pltpu.VMEM((C, D), jnp.bfloat16),
            pltpu.VMEM((C, DV), jnp.bfloat16),
            pltpu.VMEM((C * H,), jnp.float32),
            pltpu.VMEM((C * H + 8,), jnp.float32),
            pltpu.VMEM((C * H + 8,), jnp.float32),
            pltpu.VMEM((C, ACCW), jnp.float32),
            pltpu.VMEM((C, ACCW), jnp.float32),
            pltpu.SemaphoreType.DMA,
            pltpu.SemaphoreType.DMA,
            pltpu.SemaphoreType.DMA,
            pltpu.SemaphoreType.DMA,
            pltpu.SemaphoreType.DMA,
            pltpu.SemaphoreType.DMA,
            pltpu.SemaphoreType.DMA,
            pltpu.SemaphoreType.DMA,
            pltpu.SemaphoreType.DMA,
            pltpu.SemaphoreType.DMA,
        ],
    )
    return f(q, k, v, radf, src, dst, zeros80)


def _conv_kernel(y_hbm, rc_hbm, src_hbm, dst_hbm, z_hbm, out_hbm,
                 acc, sidx0, didx0, sidx1, didx1, sidx2, didx2,
                 dscat0, dscat1, yrow0, rcf0, yrow1, rcf1, msg0, msg1,
                 sy0, sr0, sy1, sr1, ssc0, ssc1):
    cid = lax.axis_index("c")
    sid = lax.axis_index("s")
    ebase = (sid * NC + cid) * EPW

    _per_subcore_rows(sid, lambda st, cnt: pltpu.sync_copy(
        z_hbm.at[pl.ds(st, cnt)], acc.at[pl.ds(st, cnt)]))
    plsc.subcore_barrier()

    bufs = ((sidx0, didx0, dscat0, yrow0, rcf0, msg0, sy0, sr0, ssc0),
            (sidx1, didx1, dscat1, yrow1, rcf1, msg1, sy1, sr1, ssc1))

    def issue(g, b):
        si, di, dsc, yr, rf, msg, sy, sr, ssc = bufs[b]
        eo = pl.multiple_of(ebase + g * CV, 8)
        pltpu.sync_copy(src_hbm.at[pl.ds(eo, CV)], si)
        pltpu.sync_copy(dst_hbm.at[pl.ds(eo, CV)], di)
        pltpu.async_copy(y_hbm.at[si], yr, sy)
        pltpu.async_copy(rc_hbm.at[pl.ds(eo, CV)], rf, sr)

    def wait(b):
        si, di, dsc, yr, rf, msg, sy, sr, ssc = bufs[b]
        pltpu.make_async_copy(y_hbm.at[si], yr, sy).wait()
        pltpu.make_async_copy(rc_hbm.at[pl.ds(0, CV)], rf, sr).wait()

    def wait_scatter(b):
        si, di, dsc, yr, rf, msg, sy, sr, ssc = bufs[b]
        pltpu.make_async_copy(msg, acc.at[dsc], ssc).wait()

    def scale(yr, rf, msg, cc):
        @plsc.parallel_loop(0, cc, unroll=2)
        def _scale(e):
            bv = plsc.load_gather(rf, [jnp.full((16,), e, jnp.int32)])
            for g in range(D // 32):
                y32 = yr[e, pl.ds(g * 32, 32)]
                ya, yb = plsc.unpack(y32, format=plsc.PackFormat.INTERLEAVED)
                msg[e, pl.ds(g * 32, 16)] = ya * bv
                msg[e, pl.ds(g * 32 + 16, 16)] = yb * bv

    def compute(b, ci):
        si, di, dsc, yr, rf, msg, sy, sr, ssc = bufs[b]

        @pl.when(ci > 0)
        def _():
            wait_scatter(b)

        scale(yr, rf, msg, CV)
        for i in range(CV // 16):
            dsc[pl.ds(i * 16, 16)] = di[pl.ds(i * 16, 16)]
        pltpu.async_copy(msg, acc.at[dsc], ssc, add=True)

    issue(0, 0)

    def _body(ci, carry):
        issue(2 * ci + 1, 1)
        wait(0)
        compute(0, ci)

        @pl.when(ci < NFULLV // 2 - 1)
        def _():
            issue(2 * ci + 2, 0)

        wait(1)
        compute(1, ci)
        return carry
    lax.fori_loop(0, NFULLV // 2, _body, 0)
    wait_scatter(0)
    wait_scatter(1)

    si, di, dsc, yr, rf, msg, sy, sr, ssc = bufs[0]
    eo = pl.multiple_of(ebase + NFULLV * CV, 8)
    pltpu.sync_copy(src_hbm.at[pl.ds(eo, REMV)], sidx2)
    pltpu.sync_copy(dst_hbm.at[pl.ds(eo, REMV)], didx2)
    pltpu.async_copy(y_hbm.at[sidx2], yr.at[pl.ds(0, REMV)], sy).wait()
    pltpu.async_copy(rc_hbm.at[pl.ds(eo, REMV)], rf.at[pl.ds(0, REMV)], sr).wait()
    scale(yr, rf, msg, REMV)
    pltpu.sync_copy(msg.at[pl.ds(0, REMV)], acc.at[didx2], add=True)

    plsc.subcore_barrier()
    _per_subcore_rows(sid, lambda st, cnt: pltpu.sync_copy(
        acc.at[pl.ds(st, cnt)], out_hbm.at[pl.ds(cid * N + st, cnt)]))


def _conv(y, rcf, src, dst, zeros128):
    f = pl.kernel(
        _conv_kernel,
        out_type=jax.ShapeDtypeStruct((NC * N, D), jnp.float32),
        mesh=_MESH,
        compiler_params=pltpu.CompilerParams(needs_layout_passes=False, use_tc_tiling_on_sc=False),
        scratch_types=[
            pltpu.VMEM_SHARED((N, D), jnp.float32),
            pltpu.VMEM((CV,), jnp.int32),
            pltpu.VMEM((CV,), jnp.int32),
            pltpu.VMEM((CV,), jnp.int32),
            pltpu.VMEM((CV,), jnp.int32),
            pltpu.VMEM((REMV,), jnp.int32),
            pltpu.VMEM((REMV,), jnp.int32),
            pltpu.VMEM((CV,), jnp.int32),
            pltpu.VMEM((CV,), jnp.int32),
            pltpu.VMEM((CV, D), jnp.bfloat16),
            pltpu.VMEM((CV,), jnp.float32),
            pltpu.VMEM((CV, D), jnp.bfloat16),
            pltpu.VMEM((CV,), jnp.float32),
            pltpu.VMEM((CV, D), jnp.float32),
            pltpu.VMEM((CV, D), jnp.float32),
            pltpu.SemaphoreType.DMA,
            pltpu.SemaphoreType.DMA,
            pltpu.SemaphoreType.DMA,
            pltpu.SemaphoreType.DMA,
            pltpu.SemaphoreType.DMA,
            pltpu.SemaphoreType.DMA,
        ],
    )
    return f(y, rcf, src, dst, zeros128)


def kernel(node_feats, edge_feats, edge_index, edge_attr, Wq, Wk, Wv, Wo,
           R1, b1, R2, gamma, Wself, Wconv, Rc1, bc1, Rc2):
    src = edge_index[0].astype(jnp.int32)
    dst = edge_index[1].astype(jnp.int32)

    ra0t, ra1t, rct = _radials(edge_feats.T, edge_attr.T, R1, b1, R2,
                               Rc1, bc1, Rc2)
    radfs = (ra0t.T.reshape(E * H), ra1t.T.reshape(E * H))
    rcf = rct.reshape(E)

    zeros80 = jnp.zeros((N, ACCW), jnp.float32)
    zeros128 = jnp.zeros((N, D), jnp.float32)

    qkp = jnp.asarray([(2 * g + s) * DK + t
                       for g in range(H // 2) for t in range(DK) for s in (0, 1)],
                      dtype=jnp.int32)
    vp = jnp.asarray([b * 32 + s * 16 + t
                      for b in range(DV // 32) for t in range(16) for s in (0, 1)],
                     dtype=jnp.int32)
    yp = jnp.asarray([b * 32 + s * 16 + t
                      for b in range(D // 32) for t in range(16) for s in (0, 1)],
                     dtype=jnp.int32)

    x = node_feats
    for l in range(LAYERS):
        q, k, v = _qkv(x, Wq[l].reshape(D, D)[:, qkp], Wk[l].reshape(D, D)[:, qkp],
                       Wv[l].reshape(D, DV)[:, vp])
        acc = _edge_attn(q, k, v, radfs[l], src, dst, zeros80)
        x = _epilogue(acc.reshape(NC, N, ACCW), Wo[l], gamma[l])

    xs, y = _finalpre(x, Wself, Wconv[:, yp])
    ca = _conv(y, rcf, src, dst, zeros128)
    return _finaladd(xs, ca.reshape(NC, N, D))


# speedup vs baseline: 84.5633x; 1.0663x over previous
import functools

import jax
import jax.numpy as jnp
from jax import lax
from jax.experimental import pallas as pl
from jax.experimental.pallas import tpu as pltpu
from jax.experimental.pallas import tpu_sc as plsc

N, E, D, H = 10000, 320000, 128, 8
DE = 4
DV = D // 2
DK = D // H
DVH = DV // H
RH = 32
LAYERS = 2

NC, NS = 2, 16
NW = NC * NS
EPW = E // NW
C = 128
NFULL = EPW // C
REM = EPW - NFULL * C
CV = 96
NFULLV = EPW // CV
REMV = EPW - NFULLV * CV
ACCW = 80
RPT = 632
RPT_LAST = N - (NS - 1) * RPT

BN = 1000
BE = 6400


def _qkv_body(x_ref, wq_ref, wk_ref, wv_ref, q_ref, k_ref, v_ref):
    x = x_ref[...]
    q_ref[...] = (jnp.dot(x, wq_ref[...], preferred_element_type=jnp.float32)
                  * 0.25).astype(jnp.bfloat16)
    k_ref[...] = jnp.dot(x, wk_ref[...],
                         preferred_element_type=jnp.float32).astype(jnp.bfloat16)
    v_ref[...] = jnp.dot(x, wv_ref[...],
                         preferred_element_type=jnp.float32).astype(jnp.bfloat16)


def _qkv(x, wq, wk, wv):
    return pl.pallas_call(
        _qkv_body,
        grid=(N // BN,),
        in_specs=[
            pl.BlockSpec((BN, D), lambda i: (i, 0)),
            pl.BlockSpec((D, D), lambda i: (0, 0)),
            pl.BlockSpec((D, D), lambda i: (0, 0)),
            pl.BlockSpec((D, DV), lambda i: (0, 0)),
        ],
        out_specs=[
            pl.BlockSpec((BN, D), lambda i: (i, 0)),
            pl.BlockSpec((BN, D), lambda i: (i, 0)),
            pl.BlockSpec((BN, DV), lambda i: (i, 0)),
        ],
        out_shape=[
            jax.ShapeDtypeStruct((N, D), jnp.bfloat16),
            jax.ShapeDtypeStruct((N, D), jnp.bfloat16),
            jax.ShapeDtypeStruct((N, DV), jnp.bfloat16),
        ],
    )(x, wq, wk, wv)


def _radial_body(fet_ref, eat_ref, r1a_ref, r2a_ref, r1b_ref, r2b_ref,
                 rc1_ref, rc2_ref, b1a_ref, b1b_ref, bc1_ref,
                 ra_ref, rb_ref, rc_ref):
    fet = fet_ref[...]
    eat = eat_ref[...]
    dist = jnp.sqrt(jnp.sum(eat * eat, axis=0, keepdims=True))
    eft = jnp.concatenate([fet, dist], axis=0)

    ha = jnp.maximum(jnp.dot(r1a_ref[...], eft, preferred_element_type=jnp.float32)
                     + b1a_ref[...], 0.0)
    ra_ref[...] = jnp.dot(r2a_ref[...], ha, preferred_element_type=jnp.float32)
    hb = jnp.maximum(jnp.dot(r1b_ref[...], eft, preferred_element_type=jnp.float32)
                     + b1b_ref[...], 0.0)
    rb_ref[...] = jnp.dot(r2b_ref[...], hb, preferred_element_type=jnp.float32)
    hc = jnp.maximum(jnp.dot(rc1_ref[...], eft, preferred_element_type=jnp.float32)
                     + bc1_ref[...], 0.0)
    rc_ref[...] = jnp.dot(rc2_ref[...], hc, preferred_element_type=jnp.float32)


def _radials(edge_feats_t, edge_attr_t, R1, b1, R2, Rc1, bc1, Rc2):
    full = lambda shape: pl.BlockSpec(shape, lambda i: (0, 0))
    return pl.pallas_call(
        _radial_body,
        grid=(E // BE,),
        in_specs=[
            pl.BlockSpec((DE, BE), lambda i: (0, i)),
            pl.BlockSpec((3, BE), lambda i: (0, i)),
            full((RH, DE + 1)), full((H, RH)),
            full((RH, DE + 1)), full((H, RH)),
            full((RH, DE + 1)), full((1, RH)),
            full((RH, 1)), full((RH, 1)), full((RH, 1)),
        ],
        out_specs=[
            pl.BlockSpec((H, BE), lambda i: (0, i)),
            pl.BlockSpec((H, BE), lambda i: (0, i)),
            pl.BlockSpec((1, BE), lambda i: (0, i)),
        ],
        out_shape=[
            jax.ShapeDtypeStruct((H, E), jnp.float32),
            jax.ShapeDtypeStruct((H, E), jnp.float32),
            jax.ShapeDtypeStruct((1, E), jnp.float32),
        ],
    )(edge_feats_t, edge_attr_t,
      R1[0].T, R2[0].T, R1[1].T, R2[1].T, Rc1.T, Rc2.T,
      b1[0].reshape(RH, 1), b1[1].reshape(RH, 1), bc1.reshape(RH, 1))


def _epilogue_body(acc_ref, wo_ref, g_ref, x_ref):
    a = acc_ref[0] + acc_ref[1]
    aggv = a[:, 0:DV]
    denom = a[:, DV:DV + H]
    r8 = lax.broadcasted_iota(jnp.int32, (H, DV), 0)
    c64 = lax.broadcasted_iota(jnp.int32, (H, DV), 1)
    sel = (c64 // DVH == r8).astype(jnp.float32)
    scale = jnp.dot(1.0 / (denom + 1e-9), sel, preferred_element_type=jnp.float32)
    x = jnp.dot(aggv * scale, wo_ref[...], preferred_element_type=jnp.float32)
    mu = jnp.mean(x, axis=1, keepdims=True)
    var = jnp.mean((x - mu) * (x - mu), axis=1, keepdims=True)
    x_ref[...] = (x - mu) / jnp.sqrt(var + 1e-5) * g_ref[...]


def _epilogue(acc, wo, gamma):
    return pl.pallas_call(
        _epilogue_body,
        grid=(N // BN,),
        in_specs=[
            pl.BlockSpec((NC, BN, ACCW), lambda i: (0, i, 0)),
            pl.BlockSpec((DV, D), lambda i: (0, 0)),
            pl.BlockSpec((1, D), lambda i: (0, 0)),
        ],
        out_specs=pl.BlockSpec((BN, D), lambda i: (i, 0)),
        out_shape=jax.ShapeDtypeStruct((N, D), jnp.float32),
    )(acc, wo, gamma.reshape(1, D))


def _finalpre_body(x_ref, ws_ref, wc_ref, xs_ref, y_ref):
    x = x_ref[...]
    xs_ref[...] = jnp.dot(x, ws_ref[...], preferred_element_type=jnp.float32)
    y_ref[...] = jnp.dot(x, wc_ref[...],
                         preferred_element_type=jnp.float32).astype(jnp.bfloat16)


def _finalpre(x, wself, wconv):
    return pl.pallas_call(
        _finalpre_body,
        grid=(N // BN,),
        in_specs=[
            pl.BlockSpec((BN, D), lambda i: (i, 0)),
            pl.BlockSpec((D, D), lambda i: (0, 0)),
            pl.BlockSpec((D, D), lambda i: (0, 0)),
        ],
        out_specs=[
            pl.BlockSpec((BN, D), lambda i: (i, 0)),
            pl.BlockSpec((BN, D), lambda i: (i, 0)),
        ],
        out_shape=[
            jax.ShapeDtypeStruct((N, D), jnp.float32),
            jax.ShapeDtypeStruct((N, D), jnp.bfloat16),
        ],
    )(x, wself, wconv)


def _finaladd_body(xs_ref, ca_ref, o_ref):
    o_ref[...] = xs_ref[...] + ca_ref[0] + ca_ref[1]


def _finaladd(xs, ca):
    return pl.pallas_call(
        _finaladd_body,
        grid=(N // BN,),
        in_specs=[
            pl.BlockSpec((BN, D), lambda i: (i, 0)),
            pl.BlockSpec((NC, BN, D), lambda i: (0, i, 0)),
        ],
        out_specs=pl.BlockSpec((BN, D), lambda i: (i, 0)),
        out_shape=jax.ShapeDtypeStruct((N, D), jnp.float32),
    )(xs, ca)


_MESH = plsc.VectorSubcoreMesh(core_axis_name="c", subcore_axis_name="s",
                               num_cores=NC, num_subcores=NS)


def _per_subcore_rows(sid, fn):
    @pl.when(sid < NS - 1)
    def _():
        fn(sid * RPT, RPT)

    @pl.when(sid == NS - 1)
    def _():
        fn((NS - 1) * RPT, RPT_LAST)


def _edge_attn_kernel(q_hbm, k_hbm, v_hbm, rad_hbm, src_hbm, dst_hbm, z_hbm,
                      out_hbm, acc,
                      sidx0, didx0, sidx1, didx1, sidx2, didx2,
                      dscat0, dscat1,
                      qrow0, krow0, vrow0, radf0,
                      qrow1, krow1, vrow1, radf1,
                      logit, pbuf, pv0, pv1,
                      sq0, sk0, sv0, sr0, sq1, sk1, sv1, sr1, ssc0, ssc1):
    cid = lax.axis_index("c")
    sid = lax.axis_index("s")
    ebase = (sid * NC + cid) * EPW

    lane = lax.broadcasted_iota(jnp.int32, (16,), 0)
    lane_lo = lane < 8
    lo_f = jnp.where(lane_lo, 1.0, 0.0).astype(jnp.float32)
    lane15 = lane == 15
    pbase = jnp.where(lane_lo, 0, 1)

    _per_subcore_rows(sid, lambda st, cnt: pltpu.sync_copy(
        z_hbm.at[pl.ds(st, cnt)], acc.at[pl.ds(st, cnt)]))

    pbuf[pl.ds(C * H - 8, 16)] = jnp.zeros((16,), jnp.float32)

    plsc.subcore_barrier()

    bufs = ((sidx0, didx0, dscat0, qrow0, krow0, vrow0, radf0, pv0,
             sq0, sk0, sv0, sr0, ssc0),
            (sidx1, didx1, dscat1, qrow1, krow1, vrow1, radf1, pv1,
             sq1, sk1, sv1, sr1, ssc1))

    def issue(g, b):
        si, di, dsc, qr, kr, vr, rf, pv, sq, sk, sv, sr, ssc = bufs[b]
        eo = pl.multiple_of(ebase + g * C, 8)
        pltpu.sync_copy(src_hbm.at[pl.ds(eo, C)], si)
        pltpu.sync_copy(dst_hbm.at[pl.ds(eo, C)], di)
        pltpu.async_copy(q_hbm.at[di], qr, sq)
        pltpu.async_copy(k_hbm.at[si], kr, sk)
        pltpu.async_copy(v_hbm.at[si], vr, sv)
        pltpu.async_copy(rad_hbm.at[pl.ds(eo * H, C * H)], rf, sr)

    def wait(b):
        si, di, dsc, qr, kr, vr, rf, pv, sq, sk, sv, sr, ssc = bufs[b]
        pltpu.make_async_copy(q_hbm.at[di], qr, sq).wait()
        pltpu.make_async_copy(k_hbm.at[si], kr, sk).wait()
        pltpu.make_async_copy(v_hbm.at[si], vr, sv).wait()
        pltpu.make_async_copy(rad_hbm.at[pl.ds(0, C * H)], rf, sr).wait()

    def wait_scatter(b):
        si, di, dsc, qr, kr, vr, rf, pv, sq, sk, sv, sr, ssc = bufs[b]
        pltpu.make_async_copy(pv, acc.at[dsc], ssc).wait()

    def dots(qr, kr, rf, cc):
        @plsc.parallel_loop(0, cc // 2, unroll=3)
        def _pair(i):
            for t in range(2):
                e = 2 * i + t
                for g in range(H // 2):
                    qb = qr[e, pl.ds(g * 32, 32)]
                    kb = kr[e, pl.ds(g * 32, 32)]
                    pa, pb = plsc.unpack(qb * kb,
                                         format=plsc.PackFormat.INTERLEAVED)
                    for h, pr in ((2 * g, pa), (2 * g + 1, pb)):
                        s = plsc.cumsum(pr)
                        plsc.store_scatter(
                            logit, [jnp.full((16,), e * H + h, jnp.int32)],
                            s, mask=lane15)
            lv = logit[pl.ds(i * 16, 16)]
            rv = rf[pl.ds(i * 16, 16)]
            pbuf[pl.ds(i * 16, 16)] = jnp.exp(lv * rv)

    def pvfill(vr, pv, cc):
        @plsc.parallel_loop(0, cc, unroll=4)
        def _pv(e):
            p16 = pbuf[pl.ds(e * H, 16)]
            pv[e, pl.ds(DV, 16)] = p16 * lo_f
            for g in range(DV // 32):
                v32 = vr[e, pl.ds(g * 32, 32)]
                va, vb = plsc.unpack(v32, format=plsc.PackFormat.INTERLEAVED)
                for j, vv in ((2 * g, va), (2 * g + 1, vb)):
                    pj = plsc.load_gather(
                        pbuf, [jnp.full((16,), e * H + 2 * j, jnp.int32) + pbase])
                    pv[e, pl.ds(j * 16, 16)] = vv * pj

    def compute(b, ci):
        si, di, dsc, qr, kr, vr, rf, pv, sq, sk, sv, sr, ssc = bufs[b]
        dots(qr, kr, rf, C)

        @pl.when(ci > 0)
        def _():
            wait_scatter(b)

        pvfill(vr, pv, C)
        for i in range(C // 16):
            dsc[pl.ds(i * 16, 16)] = di[pl.ds(i * 16, 16)]
        pltpu.async_copy(pv, acc.at[dsc], ssc, add=True)

    issue(0, 0)

    def _body(ci, carry):
        issue(2 * ci + 1, 1)
        wait(0)
        compute(0, ci)

        @pl.when(ci < NFULL // 2 - 1)
        def _():
            issue(2 * ci + 2, 0)

        wait(1)
        compute(1, ci)
        return carry
    lax.fori_loop(0, NFULL // 2, _body, 0)
    wait_scatter(0)
    wait_scatter(1)

    si, di, dsc, qr, kr, vr, rf, pv, sq, sk, sv, sr, ssc = bufs[0]
    eo = pl.multiple_of(ebase + NFULL * C, 8)
    pltpu.sync_copy(src_hbm.at[pl.ds(eo, REM)], sidx2)
    pltpu.sync_copy(dst_hbm.at[pl.ds(eo, REM)], didx2)
    pltpu.async_copy(q_hbm.at[didx2], qr.at[pl.ds(0, REM)], sq).wait()
    pltpu.async_copy(k_hbm.at[sidx2], kr.at[pl.ds(0, REM)], sk).wait()
    pltpu.async_copy(v_hbm.at[sidx2], vr.at[pl.ds(0, REM)], sv).wait()
    pltpu.async_copy(rad_hbm.at[pl.ds(eo * H, REM * H)],
                     rf.at[pl.ds(0, REM * H)], sr).wait()
    dots(qr, kr, rf, REM)
    pvfill(vr, pv, REM)
    pltpu.sync_copy(pv.at[pl.ds(0, REM)], acc.at[didx2], add=True)

    plsc.subcore_barrier()
    _per_subcore_rows(sid, lambda st, cnt: pltpu.sync_copy(
        acc.at[pl.ds(st, cnt)], out_hbm.at[pl.ds(cid * N + st, cnt)]))


def _edge_attn(q, k, v, radf, src, dst, zeros80):
    f = pl.kernel(
        _edge_attn_kernel,
        out_type=jax.ShapeDtypeStruct((NC * N, ACCW), jnp.float32),
        mesh=_MESH,
        compiler_params=pltpu.CompilerParams(needs_layout_passes=False, use_tc_tiling_on_sc=False),
        scratch_types=[
            pltpu.VMEM_SHARED((N, ACCW), jnp.float32),
            pltpu.VMEM((C,), jnp.int32),
            pltpu.VMEM((C,), jnp.int32),
            pltpu.VMEM((C,), jnp.int32),
            pltpu.VMEM((C,), jnp.int32),
            pltpu.VMEM((REM,), jnp.int32),
            pltpu.VMEM((REM,), jnp.int32),
            pltpu.VMEM((C,), jnp.int32),
            pltpu.VMEM((C,), jnp.int32),
            pltpu.VMEM((C, D), jnp.bfloat16),
            pltpu.VMEM((C, D), jnp.bfloat16),
            pltpu.VMEM((C, DV), jnp.bfloat16),
            pltpu.VMEM((C * H,), jnp.float32),
            pltpu.VMEM((C, D), jnp.bfloat16),
            pltpu.VMEM((C, D), jnp.bfloat16),
            pltpu.VMEM((C, DV), jnp.bfloat16),
            pltpu.VMEM((C * H,), jnp.float32),
            pltpu.VMEM((C * H + 8,), jnp.float32),
            pltpu.VMEM((C * H + 8,), jnp.float32),
            pltpu.VMEM((C, ACCW), jnp.float32),
            pltpu.VMEM((C, ACCW), jnp.float32),
            pltpu.SemaphoreType.DMA,
            pltpu.SemaphoreType.DMA,
            pltpu.SemaphoreType.DMA,
            pltpu.SemaphoreType.DMA,
            pltpu.SemaphoreType.DMA,
            pltpu.SemaphoreType.DMA,
            pltpu.SemaphoreType.DMA,
            pltpu.SemaphoreType.DMA,
            pltpu.SemaphoreType.DMA,
            pltpu.SemaphoreType.DMA,
        ],
    )
    return f(q, k, v, radf, src, dst, zeros80)


def _conv_kernel(y_hbm, rc_hbm, src_hbm, dst_hbm, z_hbm, out_hbm,
                 acc, sidx0, didx0, sidx1, didx1, sidx2, didx2,
                 dscat0, dscat1, yrow0, rcf0, yrow1, rcf1, msg0, msg1,
                 sy0, sr0, sy1, sr1, ssc0, ssc1):
    cid = lax.axis_index("c")
    sid = lax.axis_index("s")
    ebase = (sid * NC + cid) * EPW

    _per_subcore_rows(sid, lambda st, cnt: pltpu.sync_copy(
        z_hbm.at[pl.ds(st, cnt)], acc.at[pl.ds(st, cnt)]))
    plsc.subcore_barrier()

    bufs = ((sidx0, didx0, dscat0, yrow0, rcf0, msg0, sy0, sr0, ssc0),
            (sidx1, didx1, dscat1, yrow1, rcf1, msg1, sy1, sr1, ssc1))

    def issue(g, b):
        si, di, dsc, yr, rf, msg, sy, sr, ssc = bufs[b]
        eo = pl.multiple_of(ebase + g * CV, 8)
        pltpu.sync_copy(src_hbm.at[pl.ds(eo, CV)], si)
        pltpu.sync_copy(dst_hbm.at[pl.ds(eo, CV)], di)
        pltpu.async_copy(y_hbm.at[si], yr, sy)
        pltpu.async_copy(rc_hbm.at[pl.ds(eo, CV)], rf, sr)

    def wait(b):
        si, di, dsc, yr, rf, msg, sy, sr, ssc = bufs[b]
        pltpu.make_async_copy(y_hbm.at[si], yr, sy).wait()
        pltpu.make_async_copy(rc_hbm.at[pl.ds(0, CV)], rf, sr).wait()

    def wait_scatter(b):
        si, di, dsc, yr, rf, msg, sy, sr, ssc = bufs[b]
        pltpu.make_async_copy(msg, acc.at[dsc], ssc).wait()

    def scale(yr, rf, msg, cc):
        @plsc.parallel_loop(0, cc, unroll=2)
        def _scale(e):
            bv = plsc.load_gather(rf, [jnp.full((16,), e, jnp.int32)])
            for g in range(D // 32):
                y32 = yr[e, pl.ds(g * 32, 32)]
                ya, yb = plsc.unpack(y32, format=plsc.PackFormat.INTERLEAVED)
                msg[e, pl.ds(g * 32, 16)] = ya * bv
                msg[e, pl.ds(g * 32 + 16, 16)] = yb * bv

    def compute(b, ci):
        si, di, dsc, yr, rf, msg, sy, sr, ssc = bufs[b]

        @pl.when(ci > 0)
        def _():
            wait_scatter(b)

        scale(yr, rf, msg, CV)
        for i in range(CV // 16):
            dsc[pl.ds(i * 16, 16)] = di[pl.ds(i * 16, 16)]
        pltpu.async_copy(msg, acc.at[dsc], ssc, add=True)

    issue(0, 0)

    def _body(ci, carry):
        issue(2 * ci + 1, 1)
        wait(0)
        compute(0, ci)

        @pl.when(ci < NFULLV // 2 - 1)
        def _():
            issue(2 * ci + 2, 0)

        wait(1)
        compute(1, ci)
        return carry
    lax.fori_loop(0, NFULLV // 2, _body, 0)
    wait_scatter(0)
    wait_scatter(1)

    si, di, dsc, yr, rf, msg, sy, sr, ssc = bufs[0]
    eo = pl.multiple_of(ebase + NFULLV * CV, 8)
    pltpu.sync_copy(src_hbm.at[pl.ds(eo, REMV)], sidx2)
    pltpu.sync_copy(dst_hbm.at[pl.ds(eo, REMV)], didx2)
    pltpu.async_copy(y_hbm.at[sidx2], yr.at[pl.ds(0, REMV)], sy).wait()
    pltpu.async_copy(rc_hbm.at[pl.ds(eo, REMV)], rf.at[pl.ds(0, REMV)], sr).wait()
    scale(yr, rf, msg, REMV)
    pltpu.sync_copy(msg.at[pl.ds(0, REMV)], acc.at[didx2], add=True)

    plsc.subcore_barrier()
    _per_subcore_rows(sid, lambda st, cnt: pltpu.sync_copy(
        acc.at[pl.ds(st, cnt)], out_hbm.at[pl.ds(cid * N + st, cnt)]))


def _conv(y, rcf, src, dst, zeros128):
    f = pl.kernel(
        _conv_kernel,
        out_type=jax.ShapeDtypeStruct((NC * N, D), jnp.float32),
        mesh=_MESH,
        compiler_params=pltpu.CompilerParams(needs_layout_passes=False, use_tc_tiling_on_sc=False),
        scratch_types=[
            pltpu.VMEM_SHARED((N, D), jnp.float32),
            pltpu.VMEM((CV,), jnp.int32),
            pltpu.VMEM((CV,), jnp.int32),
            pltpu.VMEM((CV,), jnp.int32),
            pltpu.VMEM((CV,), jnp.int32),
            pltpu.VMEM((REMV,), jnp.int32),
            pltpu.VMEM((REMV,), jnp.int32),
            pltpu.VMEM((CV,), jnp.int32),
            pltpu.VMEM((CV,), jnp.int32),
            pltpu.VMEM((CV, D), jnp.bfloat16),
            pltpu.VMEM((CV,), jnp.float32),
            pltpu.VMEM((CV, D), jnp.bfloat16),
            pltpu.VMEM((CV,), jnp.float32),
            pltpu.VMEM((CV, D), jnp.float32),
            pltpu.VMEM((CV, D), jnp.float32),
            pltpu.SemaphoreType.DMA,
            pltpu.SemaphoreType.DMA,
            pltpu.SemaphoreType.DMA,
            pltpu.SemaphoreType.DMA,
            pltpu.SemaphoreType.DMA,
            pltpu.SemaphoreType.DMA,
        ],
    )
    return f(y, rcf, src, dst, zeros128)


def kernel(node_feats, edge_feats, edge_index, edge_attr, Wq, Wk, Wv, Wo,
           R1, b1, R2, gamma, Wself, Wconv, Rc1, bc1, Rc2):
    src = edge_index[0].astype(jnp.int32)
    dst = edge_index[1].astype(jnp.int32)

    ra0t, ra1t, rct = _radials(edge_feats.T, edge_attr.T, R1, b1, R2,
                               Rc1, bc1, Rc2)
    radfs = (ra0t.T.reshape(E * H), ra1t.T.reshape(E * H))
    rcf = rct.reshape(E)

    zeros80 = jnp.zeros((N, ACCW), jnp.float32)
    zeros128 = jnp.zeros((N, D), jnp.float32)

    qkp = jnp.asarray([(2 * g + s) * DK + t
                       for g in range(H // 2) for t in range(DK) for s in (0, 1)],
                      dtype=jnp.int32)
    vp = jnp.asarray([b * 32 + s * 16 + t
                      for b in range(DV // 32) for t in range(16) for s in (0, 1)],
                     dtype=jnp.int32)
    yp = jnp.asarray([b * 32 + s * 16 + t
                      for b in range(D // 32) for t in range(16) for s in (0, 1)],
                     dtype=jnp.int32)

    x = node_feats
    for l in range(LAYERS):
        q, k, v = _qkv(x, Wq[l].reshape(D, D)[:, qkp], Wk[l].reshape(D, D)[:, qkp],
                       Wv[l].reshape(D, DV)[:, vp])
        acc = _edge_attn(q, k, v, radfs[l], src, dst, zeros80)
        x = _epilogue(acc.reshape(NC, N, ACCW), Wo[l], gamma[l])

    xs, y = _finalpre(x, Wself, Wconv[:, yp])
    ca = _conv(y, rcf, src, dst, zeros128)
    return _finaladd(xs, ca.reshape(NC, N, D))
